# Initial kernel scaffold; baseline (speedup 1.0000x reference)
import functools

import jax
import jax.numpy as jnp
from jax import lax
from jax.experimental import pallas as pl
from jax.experimental.pallas import tpu as pltpu
from jax.experimental.pallas import tpu_sc as plsc

N = 10000
E = 320000
F = 128
H1, C1, D1 = 8, 16, 128
C2 = 40
HP = 16
D2P = 48

N_PAD = 10240
NCORE, NSUB = 2, 16
NTILE = NCORE * NSUB
ROWS_PER_TILE = N_PAD // NSUB
EB = 256
NBLK = 41
ET_PAD = NTILE * EB * NBLK

_f32 = jnp.float32
_mesh = plsc.VectorSubcoreMesh(core_axis_name="c", subcore_axis_name="s")


def _tc_proj1(x_pad, w1, a1s, a1d):
    def body(x_ref, w_ref, s_ref, d_ref, xp_ref, as_ref, ad_ref):
        xp = jnp.dot(x_ref[...], w_ref[...], preferred_element_type=_f32)
        xp_ref[...] = xp
        as_ref[...] = jnp.dot(xp, s_ref[...], preferred_element_type=_f32)
        ad_ref[...] = jnp.dot(xp, d_ref[...], preferred_element_type=_f32)

    return pl.pallas_call(
        body,
        out_shape=[
            jax.ShapeDtypeStruct((N_PAD, D1), _f32),
            jax.ShapeDtypeStruct((N_PAD, HP), _f32),
            jax.ShapeDtypeStruct((N_PAD, HP), _f32),
        ],
    )(x_pad, w1, a1s, a1d)


def _tc_inv1(den_p):
    def body(d_ref, o_ref):
        o_ref[...] = 1.0 / (d_ref[0] + d_ref[1] + 1e-16)

    return pl.pallas_call(
        body, out_shape=jax.ShapeDtypeStruct((N_PAD, HP), _f32)
    )(den_p)


def _tc_proj2(out1_p, b1, w2p, a2):
    def body(p_ref, b_ref, w_ref, a_ref, xp_ref, al_ref):
        h = p_ref[0] + p_ref[1] + b_ref[...]
        xp = jnp.dot(h, w_ref[...], preferred_element_type=_f32)
        xp_ref[...] = xp
        al_ref[...] = jnp.dot(xp, a_ref[...], preferred_element_type=_f32)

    return pl.pallas_call(
        body,
        out_shape=[
            jax.ShapeDtypeStruct((N_PAD, D2P), _f32),
            jax.ShapeDtypeStruct((N_PAD, HP), _f32),
        ],
    )(out1_p, b1, w2p, a2)


def _tc_inv2(den2_p):
    def body(d_ref, o_ref):
        o_ref[...] = 1.0 / (jnp.sum(d_ref[...], axis=0) + 1e-16)

    return pl.pallas_call(
        body, out_shape=jax.ShapeDtypeStruct((80, 128), _f32)
    )(den2_p)


def _tc_final(out2_p, b2p):
    def body(p_ref, b_ref, o_ref):
        o_ref[...] = p_ref[0] + p_ref[1] + b_ref[...]

    return pl.pallas_call(
        body, out_shape=jax.ShapeDtypeStruct((N_PAD, D2P), _f32)
    )(out2_p, b2p)


def _sc_pass1_l1(src, dst, as1, ad1):
    @functools.partial(
        pl.kernel, mesh=_mesh,
        out_type=[
            jax.ShapeDtypeStruct((ET_PAD, HP), _f32),
            jax.ShapeDtypeStruct((NCORE, N_PAD, HP), _f32),
        ],
        scratch_types=[
            pltpu.VMEM_SHARED((N_PAD, HP), _f32),
            pltpu.VMEM((EB,), jnp.int32),
            pltpu.VMEM((EB,), jnp.int32),
            pltpu.VMEM((EB, HP), _f32),
            pltpu.VMEM((EB, HP), _f32),
            pltpu.VMEM((EB, HP), _f32),
        ],
    )
    def k(src_hbm, dst_hbm, as_hbm, ad_hbm, ex_hbm, den_hbm,
          den_sh, sidx, didx, ga, gb, exb):
        c = lax.axis_index("c")
        s = lax.axis_index("s")
        wid = c * NSUB + s

        @pl.loop(0, EB)
        def _(r):
            exb.at[r][...] = jnp.zeros((HP,), _f32)

        row0 = s * ROWS_PER_TILE
        pltpu.sync_copy(exb, den_sh.at[pl.ds(row0, EB)])
        pltpu.sync_copy(exb, den_sh.at[pl.ds(row0 + EB, EB)])
        pltpu.sync_copy(exb.at[pl.ds(0, ROWS_PER_TILE - 2 * EB)],
                        den_sh.at[pl.ds(row0 + 2 * EB, ROWS_PER_TILE - 2 * EB)])
        plsc.subcore_barrier()

        @pl.loop(0, NBLK)
        def _(blk):
            eoff = (wid * NBLK + blk) * EB
            pltpu.sync_copy(src_hbm.at[pl.ds(eoff, EB)], sidx)
            pltpu.sync_copy(dst_hbm.at[pl.ds(eoff, EB)], didx)
            pltpu.sync_copy(as_hbm.at[sidx], ga)
            pltpu.sync_copy(ad_hbm.at[didx], gb)

            @pl.loop(0, EB)
            def _(e):
                a = ga.at[e][...] + gb.at[e][...]
                exb.at[e][...] = jnp.exp(jnp.maximum(a, 0.2 * a))

            pltpu.sync_copy(exb, den_sh.at[didx], add=True)
            pltpu.sync_copy(exb, ex_hbm.at[pl.ds(eoff, EB)])

        plsc.subcore_barrier()
        pltpu.sync_copy(den_sh.at[pl.ds(row0, ROWS_PER_TILE)],
                        den_hbm.at[c].at[pl.ds(row0, ROWS_PER_TILE)])

    return k(src, dst, as1, ad1)


def _sc_pass2_l1(src, dst, ex1, inv1, xp1):
    @functools.partial(
        pl.kernel, mesh=_mesh,
        out_type=jax.ShapeDtypeStruct((NCORE, N_PAD, D1), _f32),
        scratch_types=[
            pltpu.VMEM_SHARED((N_PAD, D1), _f32),
            pltpu.VMEM((EB,), jnp.int32),
            pltpu.VMEM((EB,), jnp.int32),
            pltpu.VMEM((EB, HP), _f32),
            pltpu.VMEM((EB, HP), _f32),
            pltpu.VMEM((EB, D1), _f32),
        ],
    )
    def k(src_hbm, dst_hbm, ex_hbm, inv_hbm, xp_hbm, out_hbm,
          out_sh, sidx, didx, exb, ivb, xsb):
        c = lax.axis_index("c")
        s = lax.axis_index("s")
        wid = c * NSUB + s

        @pl.loop(0, EB)
        def _(r):
            row = xsb.at[r]
            for h in range(D1 // 16):
                row[pl.ds(h * 16, 16)] = jnp.zeros((16,), _f32)

        row0 = s * ROWS_PER_TILE
        pltpu.sync_copy(xsb, out_sh.at[pl.ds(row0, EB)])
        pltpu.sync_copy(xsb, out_sh.at[pl.ds(row0 + EB, EB)])
        pltpu.sync_copy(xsb.at[pl.ds(0, ROWS_PER_TILE - 2 * EB)],
                        out_sh.at[pl.ds(row0 + 2 * EB, ROWS_PER_TILE - 2 * EB)])
        plsc.subcore_barrier()

        @pl.loop(0, NBLK)
        def _(blk):
            eoff = (wid * NBLK + blk) * EB
            pltpu.sync_copy(src_hbm.at[pl.ds(eoff, EB)], sidx)
            pltpu.sync_copy(dst_hbm.at[pl.ds(eoff, EB)], didx)
            pltpu.sync_copy(ex_hbm.at[pl.ds(eoff, EB)], exb)
            pltpu.sync_copy(inv_hbm.at[didx], ivb)
            pltpu.sync_copy(xp_hbm.at[sidx], xsb)

            @pl.loop(0, EB)
            def _(e):
                exb.at[e][...] = exb.at[e][...] * ivb.at[e][...]
                row = xsb.at[e]
                for h in range(H1):
                    av = jnp.full((16,), exb[e, h], _f32)
                    row[pl.ds(h * 16, 16)] = row[pl.ds(h * 16, 16)] * av

            pltpu.sync_copy(xsb, out_sh.at[didx], add=True)

        plsc.subcore_barrier()
        pltpu.sync_copy(out_sh.at[pl.ds(row0, ROWS_PER_TILE)],
                        out_hbm.at[c].at[pl.ds(row0, ROWS_PER_TILE)])

    return k(src, dst, ex1, inv1, xp1)


def _sc_pass1_l2(src, dst, as2, ad2):
    @functools.partial(
        pl.kernel, mesh=_mesh,
        out_type=[
            jax.ShapeDtypeStruct((ET_PAD,), _f32),
            jax.ShapeDtypeStruct((NTILE, N_PAD), _f32),
        ],
        scratch_types=[
            pltpu.VMEM((N_PAD,), _f32),
            pltpu.VMEM((N_PAD,), _f32),
            pltpu.VMEM((N_PAD,), _f32),
            pltpu.VMEM((EB,), jnp.int32),
            pltpu.VMEM((EB,), jnp.int32),
            pltpu.VMEM((EB,), _f32),
        ],
    )
    def k(src_hbm, dst_hbm, as_hbm, ad_hbm, ex_hbm, den_hbm,
          tabs, tabd, den, sidx, didx, exb):
        c = lax.axis_index("c")
        s = lax.axis_index("s")
        wid = c * NSUB + s

        pltpu.sync_copy(as_hbm, tabs)
        pltpu.sync_copy(ad_hbm, tabd)

        @pl.loop(0, N_PAD, step=16)
        def _(i):
            den[pl.ds(i, 16)] = jnp.zeros((16,), _f32)

        @pl.loop(0, NBLK)
        def _(blk):
            eoff = (wid * NBLK + blk) * EB
            pltpu.sync_copy(src_hbm.at[pl.ds(eoff, EB)], sidx)
            pltpu.sync_copy(dst_hbm.at[pl.ds(eoff, EB)], didx)

            @pl.loop(0, EB, step=16)
            def _(e):
                s16 = sidx[pl.ds(e, 16)]
                d16 = didx[pl.ds(e, 16)]
                a = plsc.load_gather(tabs, [s16]) + plsc.load_gather(tabd, [d16])
                ex = jnp.exp(jnp.maximum(a, 0.2 * a))
                exb[pl.ds(e, 16)] = ex
                plsc.addupdate_scatter(den, [d16], ex)

            pltpu.sync_copy(exb, ex_hbm.at[pl.ds(eoff, EB)])

        pltpu.sync_copy(den, den_hbm.at[wid])

    return k(src, dst, as2, ad2)


def _sc_pass2_l2(src, dst, ex2, inv2, xp2):
    @functools.partial(
        pl.kernel, mesh=_mesh,
        out_type=jax.ShapeDtypeStruct((NCORE, N_PAD, D2P), _f32),
        scratch_types=[
            pltpu.VMEM_SHARED((N_PAD, D2P), _f32),
            pltpu.VMEM((N_PAD,), _f32),
            pltpu.VMEM((EB,), jnp.int32),
            pltpu.VMEM((EB,), jnp.int32),
            pltpu.VMEM((EB,), _f32),
            pltpu.VMEM((EB, D2P), _f32),
        ],
    )
    def k(src_hbm, dst_hbm, ex_hbm, inv_hbm, xp_hbm, out_hbm,
          out_sh, tabi, sidx, didx, exb, xsb):
        c = lax.axis_index("c")
        s = lax.axis_index("s")
        wid = c * NSUB + s

        pltpu.sync_copy(inv_hbm, tabi)

        @pl.loop(0, EB)
        def _(r):
            row = xsb.at[r]
            for h in range(D2P // 16):
                row[pl.ds(h * 16, 16)] = jnp.zeros((16,), _f32)

        row0 = s * ROWS_PER_TILE
        pltpu.sync_copy(xsb, out_sh.at[pl.ds(row0, EB)])
        pltpu.sync_copy(xsb, out_sh.at[pl.ds(row0 + EB, EB)])
        pltpu.sync_copy(xsb.at[pl.ds(0, ROWS_PER_TILE - 2 * EB)],
                        out_sh.at[pl.ds(row0 + 2 * EB, ROWS_PER_TILE - 2 * EB)])
        plsc.subcore_barrier()

        @pl.loop(0, NBLK)
        def _(blk):
            eoff = (wid * NBLK + blk) * EB
            pltpu.sync_copy(src_hbm.at[pl.ds(eoff, EB)], sidx)
            pltpu.sync_copy(dst_hbm.at[pl.ds(eoff, EB)], didx)
            pltpu.sync_copy(ex_hbm.at[pl.ds(eoff, EB)], exb)
            pltpu.sync_copy(xp_hbm.at[sidx], xsb)

            @pl.loop(0, EB, step=16)
            def _(e):
                d16 = didx[pl.ds(e, 16)]
                iv = plsc.load_gather(tabi, [d16])
                exb[pl.ds(e, 16)] = exb[pl.ds(e, 16)] * iv

            @pl.loop(0, EB)
            def _(e):
                av = jnp.full((16,), exb[e], _f32)
                row = xsb.at[e]
                for h in range(D2P // 16):
                    row[pl.ds(h * 16, 16)] = row[pl.ds(h * 16, 16)] * av

            pltpu.sync_copy(xsb, out_sh.at[didx], add=True)

        plsc.subcore_barrier()
        pltpu.sync_copy(out_sh.at[pl.ds(row0, ROWS_PER_TILE)],
                        out_hbm.at[c].at[pl.ds(row0, ROWS_PER_TILE)])

    return k(src, dst, ex2, inv2, xp2)


def kernel(x, edge_index, W1, att_src1, att_dst1, b1, W2, att_src2, att_dst2,
           b2):
    i32 = jnp.int32
    loop = jnp.arange(N, dtype=i32)
    pad = jnp.full((ET_PAD - E - N,), N, dtype=i32)
    src = jnp.concatenate([edge_index[0], loop, pad])
    dst = jnp.concatenate([edge_index[1], loop, pad])

    x_pad = jnp.pad(x, ((0, N_PAD - N), (0, 0)))
    a1s = jnp.pad((att_src1[:, :, None] * jnp.eye(H1, dtype=_f32)[:, None, :])
                  .reshape(D1, H1), ((0, 0), (0, HP - H1)))
    a1d = jnp.pad((att_dst1[:, :, None] * jnp.eye(H1, dtype=_f32)[:, None, :])
                  .reshape(D1, H1), ((0, 0), (0, HP - H1)))
    w2p = jnp.pad(W2, ((0, 0), (0, D2P - C2)))
    a2 = jnp.zeros((D2P, HP), _f32)
    a2 = a2.at[:C2, 0].set(att_src2[0]).at[:C2, 1].set(att_dst2[0])
    b1r = b1.reshape(1, D1)
    b2p = jnp.pad(b2, (0, D2P - C2)).reshape(1, D2P)

    xp1, as1, ad1 = _tc_proj1(x_pad, W1, a1s, a1d)
    ex1, den1p = _sc_pass1_l1(src, dst, as1, ad1)
    inv1 = _tc_inv1(den1p)
    out1p = _sc_pass2_l1(src, dst, ex1, inv1, xp1)

    xp2, aall = _tc_proj2(out1p, b1r, w2p, a2)
    as2 = aall[:, 0]
    ad2 = aall[:, 1]
    ex2, den2p = _sc_pass1_l2(src, dst, as2, ad2)
    inv2 = _tc_inv2(den2p.reshape(NTILE, 80, 128)).reshape(N_PAD)
    out2p = _sc_pass2_l2(src, dst, ex2, inv2, xp2)

    res = _tc_final(out2p, b2p)
    return res[:N, :C2]


# speedup vs baseline: 39.2857x; 39.2857x over previous
import functools

import jax
import jax.numpy as jnp
from jax import lax
from jax.experimental import pallas as pl
from jax.experimental.pallas import tpu as pltpu
from jax.experimental.pallas import tpu_sc as plsc

N = 10000
E = 320000
F = 128
H1, C1, D1 = 8, 16, 128
C2 = 40
HP = 16
D2P = 48

N_PAD = 10240
NCORE, NSUB = 2, 16
NTILE = NCORE * NSUB
ROWS_PER_TILE = N_PAD // NSUB
EB = 256
NBLK = 41
ET_PAD = NTILE * EB * NBLK

_f32 = jnp.float32
_mesh = plsc.VectorSubcoreMesh(core_axis_name="c", subcore_axis_name="s")
_sc_params = pltpu.CompilerParams(use_tc_tiling_on_sc=False,
                                  needs_layout_passes=False)


def _tc_proj1(x_pad, w1, a1s, a1d):
    def body(x_ref, w_ref, s_ref, d_ref, xp_ref, as_ref, ad_ref):
        xp = jnp.dot(x_ref[...], w_ref[...], preferred_element_type=_f32)
        xp_ref[...] = xp
        as_ref[...] = jnp.dot(xp, s_ref[...], preferred_element_type=_f32)
        ad_ref[...] = jnp.dot(xp, d_ref[...], preferred_element_type=_f32)

    return pl.pallas_call(
        body,
        out_shape=[
            jax.ShapeDtypeStruct((N_PAD, D1), _f32),
            jax.ShapeDtypeStruct((N_PAD, HP), _f32),
            jax.ShapeDtypeStruct((N_PAD, HP), _f32),
        ],
    )(x_pad, w1, a1s, a1d)


def _tc_inv1(den_p):
    def body(d_ref, o_ref):
        o_ref[...] = 1.0 / (d_ref[0] + d_ref[1] + 1e-16)

    return pl.pallas_call(
        body, out_shape=jax.ShapeDtypeStruct((N_PAD, HP), _f32)
    )(den_p)


def _tc_proj2(out1_p, b1, w2p, a2):
    def body(p_ref, b_ref, w_ref, a_ref, xp_ref, al_ref):
        h = p_ref[0] + p_ref[1] + b_ref[...]
        xp = jnp.dot(h, w_ref[...], preferred_element_type=_f32)
        xp_ref[...] = xp
        al_ref[...] = jnp.dot(xp, a_ref[...], preferred_element_type=_f32)

    return pl.pallas_call(
        body,
        out_shape=[
            jax.ShapeDtypeStruct((N_PAD, D2P), _f32),
            jax.ShapeDtypeStruct((N_PAD, HP), _f32),
        ],
    )(out1_p, b1, w2p, a2)


def _tc_inv2(den2_p):
    def body(d_ref, o_ref):
        o_ref[...] = 1.0 / (jnp.sum(d_ref[...], axis=0) + 1e-16)

    return pl.pallas_call(
        body, out_shape=jax.ShapeDtypeStruct((80, 128), _f32)
    )(den2_p)


def _tc_final(out2_p, b2p):
    def body(p_ref, b_ref, o_ref):
        o_ref[...] = p_ref[0] + p_ref[1] + b_ref[...]

    return pl.pallas_call(
        body, out_shape=jax.ShapeDtypeStruct((N_PAD, D2P), _f32)
    )(out2_p, b2p)


def _sc_pass1_l1(src, dst, as1, ad1):
    @functools.partial(
        pl.kernel, mesh=_mesh, compiler_params=_sc_params,
        out_type=[
            jax.ShapeDtypeStruct((ET_PAD, HP), _f32),
            jax.ShapeDtypeStruct((NCORE, N_PAD, HP), _f32),
        ],
        scratch_types=[
            pltpu.VMEM_SHARED((N_PAD, HP), _f32),
            pltpu.VMEM((EB,), jnp.int32),
            pltpu.VMEM((EB,), jnp.int32),
            pltpu.VMEM((EB, HP), _f32),
            pltpu.VMEM((EB, HP), _f32),
            pltpu.VMEM((EB, HP), _f32),
        ],
    )
    def k(src_hbm, dst_hbm, as_hbm, ad_hbm, ex_hbm, den_hbm,
          den_sh, sidx, didx, ga, gb, exb):
        c = lax.axis_index("c")
        s = lax.axis_index("s")
        wid = c * NSUB + s

        @pl.loop(0, EB)
        def _(r):
            exb.at[r][...] = jnp.zeros((HP,), _f32)

        row0 = s * ROWS_PER_TILE
        pltpu.sync_copy(exb, den_sh.at[pl.ds(row0, EB)])
        pltpu.sync_copy(exb, den_sh.at[pl.ds(row0 + EB, EB)])
        pltpu.sync_copy(exb.at[pl.ds(0, ROWS_PER_TILE - 2 * EB)],
                        den_sh.at[pl.ds(row0 + 2 * EB, ROWS_PER_TILE - 2 * EB)])
        plsc.subcore_barrier()

        @pl.loop(0, NBLK)
        def _(blk):
            eoff = (wid * NBLK + blk) * EB
            pltpu.sync_copy(src_hbm.at[pl.ds(eoff, EB)], sidx)
            pltpu.sync_copy(dst_hbm.at[pl.ds(eoff, EB)], didx)
            pltpu.sync_copy(as_hbm.at[sidx], ga)
            pltpu.sync_copy(ad_hbm.at[didx], gb)

            @pl.loop(0, EB)
            def _(e):
                a = ga.at[e][...] + gb.at[e][...]
                exb.at[e][...] = jnp.exp(jnp.maximum(a, 0.2 * a))

            pltpu.sync_copy(exb, den_sh.at[didx], add=True)
            pltpu.sync_copy(exb, ex_hbm.at[pl.ds(eoff, EB)])

        plsc.subcore_barrier()
        pltpu.sync_copy(den_sh.at[pl.ds(row0, ROWS_PER_TILE)],
                        den_hbm.at[c].at[pl.ds(row0, ROWS_PER_TILE)])

    return k(src, dst, as1, ad1)


def _sc_pass2_l1(src, dst, ex1, inv1, xp1):
    @functools.partial(
        pl.kernel, mesh=_mesh, compiler_params=_sc_params,
        out_type=jax.ShapeDtypeStruct((NCORE, N_PAD, D1), _f32),
        scratch_types=[
            pltpu.VMEM_SHARED((N_PAD, D1), _f32),
            pltpu.VMEM((EB,), jnp.int32),
            pltpu.VMEM((EB,), jnp.int32),
            pltpu.VMEM((EB, HP), _f32),
            pltpu.VMEM((EB, HP), _f32),
            pltpu.VMEM((EB, D1), _f32),
        ],
    )
    def k(src_hbm, dst_hbm, ex_hbm, inv_hbm, xp_hbm, out_hbm,
          out_sh, sidx, didx, exb, ivb, xsb):
        c = lax.axis_index("c")
        s = lax.axis_index("s")
        wid = c * NSUB + s

        @pl.loop(0, EB)
        def _(r):
            row = xsb.at[r]
            for h in range(D1 // 16):
                row[pl.ds(h * 16, 16)] = jnp.zeros((16,), _f32)

        row0 = s * ROWS_PER_TILE
        pltpu.sync_copy(xsb, out_sh.at[pl.ds(row0, EB)])
        pltpu.sync_copy(xsb, out_sh.at[pl.ds(row0 + EB, EB)])
        pltpu.sync_copy(xsb.at[pl.ds(0, ROWS_PER_TILE - 2 * EB)],
                        out_sh.at[pl.ds(row0 + 2 * EB, ROWS_PER_TILE - 2 * EB)])
        plsc.subcore_barrier()

        @pl.loop(0, NBLK)
        def _(blk):
            eoff = (wid * NBLK + blk) * EB
            pltpu.sync_copy(src_hbm.at[pl.ds(eoff, EB)], sidx)
            pltpu.sync_copy(dst_hbm.at[pl.ds(eoff, EB)], didx)
            pltpu.sync_copy(ex_hbm.at[pl.ds(eoff, EB)], exb)
            pltpu.sync_copy(inv_hbm.at[didx], ivb)
            pltpu.sync_copy(xp_hbm.at[sidx], xsb)

            @pl.loop(0, EB)
            def _(e):
                arow = exb.at[e][...] * ivb.at[e][...]
                row = xsb.at[e]
                for h in range(H1):
                    av = jnp.full((16,), arow[h], _f32)
                    row[pl.ds(h * 16, 16)] = row[pl.ds(h * 16, 16)] * av

            pltpu.sync_copy(xsb, out_sh.at[didx], add=True)

        plsc.subcore_barrier()
        pltpu.sync_copy(out_sh.at[pl.ds(row0, ROWS_PER_TILE)],
                        out_hbm.at[c].at[pl.ds(row0, ROWS_PER_TILE)])

    return k(src, dst, ex1, inv1, xp1)


def _sc_pass1_l2(src, dst, as2, ad2):
    @functools.partial(
        pl.kernel, mesh=_mesh, compiler_params=_sc_params,
        out_type=[
            jax.ShapeDtypeStruct((ET_PAD,), _f32),
            jax.ShapeDtypeStruct((NTILE, N_PAD), _f32),
        ],
        scratch_types=[
            pltpu.VMEM((N_PAD,), _f32),
            pltpu.VMEM((N_PAD,), _f32),
            pltpu.VMEM((N_PAD,), _f32),
            pltpu.VMEM((EB,), jnp.int32),
            pltpu.VMEM((EB,), jnp.int32),
            pltpu.VMEM((EB,), _f32),
        ],
    )
    def k(src_hbm, dst_hbm, as_hbm, ad_hbm, ex_hbm, den_hbm,
          tabs, tabd, den, sidx, didx, exb):
        c = lax.axis_index("c")
        s = lax.axis_index("s")
        wid = c * NSUB + s

        pltpu.sync_copy(as_hbm, tabs)
        pltpu.sync_copy(ad_hbm, tabd)

        @pl.loop(0, N_PAD, step=16)
        def _(i):
            den[pl.ds(i, 16)] = jnp.zeros((16,), _f32)

        @pl.loop(0, NBLK)
        def _(blk):
            eoff = (wid * NBLK + blk) * EB
            pltpu.sync_copy(src_hbm.at[pl.ds(eoff, EB)], sidx)
            pltpu.sync_copy(dst_hbm.at[pl.ds(eoff, EB)], didx)

            @pl.loop(0, EB, step=16)
            def _(e):
                s16 = sidx[pl.ds(e, 16)]
                d16 = didx[pl.ds(e, 16)]
                a = plsc.load_gather(tabs, [s16]) + plsc.load_gather(tabd, [d16])
                ex = jnp.exp(jnp.maximum(a, 0.2 * a))
                exb[pl.ds(e, 16)] = ex
                plsc.addupdate_scatter(den, [d16], ex)

            pltpu.sync_copy(exb, ex_hbm.at[pl.ds(eoff, EB)])

        pltpu.sync_copy(den, den_hbm.at[wid])

    return k(src, dst, as2, ad2)


def _sc_pass2_l2(src, dst, ex2, inv2, xp2):
    @functools.partial(
        pl.kernel, mesh=_mesh, compiler_params=_sc_params,
        out_type=jax.ShapeDtypeStruct((NCORE, N_PAD, D2P), _f32),
        scratch_types=[
            pltpu.VMEM_SHARED((N_PAD, D2P), _f32),
            pltpu.VMEM((N_PAD,), _f32),
            pltpu.VMEM((EB,), jnp.int32),
            pltpu.VMEM((EB,), jnp.int32),
            pltpu.VMEM((EB,), _f32),
            pltpu.VMEM((EB, D2P), _f32),
        ],
    )
    def k(src_hbm, dst_hbm, ex_hbm, inv_hbm, xp_hbm, out_hbm,
          out_sh, tabi, sidx, didx, exb, xsb):
        c = lax.axis_index("c")
        s = lax.axis_index("s")
        wid = c * NSUB + s

        pltpu.sync_copy(inv_hbm, tabi)

        @pl.loop(0, EB)
        def _(r):
            row = xsb.at[r]
            for h in range(D2P // 16):
                row[pl.ds(h * 16, 16)] = jnp.zeros((16,), _f32)

        row0 = s * ROWS_PER_TILE
        pltpu.sync_copy(xsb, out_sh.at[pl.ds(row0, EB)])
        pltpu.sync_copy(xsb, out_sh.at[pl.ds(row0 + EB, EB)])
        pltpu.sync_copy(xsb.at[pl.ds(0, ROWS_PER_TILE - 2 * EB)],
                        out_sh.at[pl.ds(row0 + 2 * EB, ROWS_PER_TILE - 2 * EB)])
        plsc.subcore_barrier()

        @pl.loop(0, NBLK)
        def _(blk):
            eoff = (wid * NBLK + blk) * EB
            pltpu.sync_copy(src_hbm.at[pl.ds(eoff, EB)], sidx)
            pltpu.sync_copy(dst_hbm.at[pl.ds(eoff, EB)], didx)
            pltpu.sync_copy(ex_hbm.at[pl.ds(eoff, EB)], exb)
            pltpu.sync_copy(xp_hbm.at[sidx], xsb)

            @pl.loop(0, EB, step=16)
            def _(e):
                d16 = didx[pl.ds(e, 16)]
                a16 = exb[pl.ds(e, 16)] * plsc.load_gather(tabi, [d16])
                for j in range(16):
                    av = jnp.full((16,), a16[j], _f32)
                    row = xsb.at[e + j]
                    for h in range(D2P // 16):
                        row[pl.ds(h * 16, 16)] = row[pl.ds(h * 16, 16)] * av

            pltpu.sync_copy(xsb, out_sh.at[didx], add=True)

        plsc.subcore_barrier()
        pltpu.sync_copy(out_sh.at[pl.ds(row0, ROWS_PER_TILE)],
                        out_hbm.at[c].at[pl.ds(row0, ROWS_PER_TILE)])

    return k(src, dst, ex2, inv2, xp2)


def kernel(x, edge_index, W1, att_src1, att_dst1, b1, W2, att_src2, att_dst2,
           b2):
    i32 = jnp.int32
    loop = jnp.arange(N, dtype=i32)
    pad = jnp.full((ET_PAD - E - N,), N, dtype=i32)
    src = jnp.concatenate([edge_index[0], loop, pad])
    dst = jnp.concatenate([edge_index[1], loop, pad])

    x_pad = jnp.pad(x, ((0, N_PAD - N), (0, 0)))
    a1s = jnp.pad((att_src1[:, :, None] * jnp.eye(H1, dtype=_f32)[:, None, :])
                  .reshape(D1, H1), ((0, 0), (0, HP - H1)))
    a1d = jnp.pad((att_dst1[:, :, None] * jnp.eye(H1, dtype=_f32)[:, None, :])
                  .reshape(D1, H1), ((0, 0), (0, HP - H1)))
    w2p = jnp.pad(W2, ((0, 0), (0, D2P - C2)))
    a2 = jnp.zeros((D2P, HP), _f32)
    a2 = a2.at[:C2, 0].set(att_src2[0]).at[:C2, 1].set(att_dst2[0])
    b1r = b1.reshape(1, D1)
    b2p = jnp.pad(b2, (0, D2P - C2)).reshape(1, D2P)

    xp1, as1, ad1 = _tc_proj1(x_pad, W1, a1s, a1d)
    ex1, den1p = _sc_pass1_l1(src, dst, as1, ad1)
    inv1 = _tc_inv1(den1p)
    out1p = _sc_pass2_l1(src, dst, ex1, inv1, xp1)

    xp2, aall = _tc_proj2(out1p, b1r, w2p, a2)
    as2 = aall[:, 0]
    ad2 = aall[:, 1]
    ex2, den2p = _sc_pass1_l2(src, dst, as2, ad2)
    inv2 = _tc_inv2(den2p.reshape(NTILE, 80, 128)).reshape(N_PAD)
    out2p = _sc_pass2_l2(src, dst, ex2, inv2, xp2)

    res = _tc_final(out2p, b2p)
    return res[:N, :C2]


# speedup vs baseline: 90.9454x; 2.3150x over previous
import functools

import jax
import jax.numpy as jnp
from jax import lax
from jax.experimental import pallas as pl
from jax.experimental.pallas import tpu as pltpu
from jax.experimental.pallas import tpu_sc as plsc

N = 10000
E = 320000
F = 128
H1, C1, D1 = 8, 16, 128
C2 = 40
HP = 16
D2P = 48

N_PAD = 10240
NCORE, NSUB = 2, 16
NTILE = NCORE * NSUB
ROWS_PER_TILE = N_PAD // NSUB
EB = 256
NBLK = 41
ET_PAD = NTILE * EB * NBLK

_f32 = jnp.float32
_mesh = plsc.VectorSubcoreMesh(core_axis_name="c", subcore_axis_name="s")
_sc_params = pltpu.CompilerParams(use_tc_tiling_on_sc=False,
                                  needs_layout_passes=False)


def _tc_proj1(x_pad, w1, a1s, a1d):
    def body(x_ref, w_ref, s_ref, d_ref, xpa_ref, xpb_ref, as_ref, ad_ref):
        xp = jnp.dot(x_ref[...], w_ref[...], preferred_element_type=_f32)
        xpa_ref[...] = xp[:, :D1 // 2]
        xpb_ref[...] = xp[:, D1 // 2:]
        as_ref[...] = jnp.dot(xp, s_ref[...], preferred_element_type=_f32)
        ad_ref[...] = jnp.dot(xp, d_ref[...], preferred_element_type=_f32)

    return pl.pallas_call(
        body,
        out_shape=[
            jax.ShapeDtypeStruct((N_PAD, D1 // 2), _f32),
            jax.ShapeDtypeStruct((N_PAD, D1 // 2), _f32),
            jax.ShapeDtypeStruct((N_PAD, HP), _f32),
            jax.ShapeDtypeStruct((N_PAD, HP), _f32),
        ],
    )(x_pad, w1, a1s, a1d)


def _tc_inv1(den_p):
    def body(d_ref, o_ref):
        o_ref[...] = 1.0 / (d_ref[0] + d_ref[1] + 1e-16)

    return pl.pallas_call(
        body, out_shape=jax.ShapeDtypeStruct((N_PAD, HP), _f32)
    )(den_p)


def _tc_proj2(out1_pa, out1_pb, b1, w2p, a2):
    def body(pa_ref, pb_ref, b_ref, w_ref, a_ref, xp_ref, al_ref):
        h = jnp.concatenate([pa_ref[0] + pa_ref[1], pb_ref[0] + pb_ref[1]],
                            axis=1) + b_ref[...]
        xp = jnp.dot(h, w_ref[...], preferred_element_type=_f32)
        xp_ref[...] = xp
        al_ref[...] = jnp.dot(xp, a_ref[...], preferred_element_type=_f32)

    return pl.pallas_call(
        body,
        out_shape=[
            jax.ShapeDtypeStruct((N_PAD, D2P), _f32),
            jax.ShapeDtypeStruct((N_PAD, HP), _f32),
        ],
    )(out1_pa, out1_pb, b1, w2p, a2)


def _tc_inv2(den2_p):
    def body(d_ref, o_ref):
        o_ref[...] = 1.0 / (jnp.sum(d_ref[...], axis=0) + 1e-16)

    return pl.pallas_call(
        body, out_shape=jax.ShapeDtypeStruct((80, 128), _f32)
    )(den2_p)


def _tc_final(out2_p, b2p):
    def body(p_ref, b_ref, o_ref):
        o_ref[...] = p_ref[0] + p_ref[1] + b_ref[...]

    return pl.pallas_call(
        body, out_shape=jax.ShapeDtypeStruct((N_PAD, D2P), _f32)
    )(out2_p, b2p)


def _zero_rows(zbuf, dst_sh, row0):
    pltpu.sync_copy(zbuf, dst_sh.at[pl.ds(row0, EB)])
    pltpu.sync_copy(zbuf, dst_sh.at[pl.ds(row0 + EB, EB)])
    pltpu.sync_copy(zbuf.at[pl.ds(0, ROWS_PER_TILE - 2 * EB)],
                    dst_sh.at[pl.ds(row0 + 2 * EB, ROWS_PER_TILE - 2 * EB)])


def _stage_idx(ibt, sidx, didx):
    def stage(j, q):
        srow = ibt.at[j, 0]
        drow = ibt.at[j, 1]

        @pl.loop(0, EB, step=16)
        def _(t):
            sidx[q][pl.ds(t, 16)] = srow[pl.ds(t, 16)]
            didx[q][pl.ds(t, 16)] = drow[pl.ds(t, 16)]

    return stage


def _pipeline(stage_idx, issue_gathers, wait_gathers, compute, issue_outs,
              wait_outs):
    stage_idx(0, 0)
    issue_gathers(0, 0)

    def step(i, p):
        q = 1 - p
        wait_gathers(p)

        @pl.when(i + 1 < NBLK)
        def _():
            @pl.when(i >= 1)
            def _():
                wait_outs(q)
            stage_idx(i + 1, q)
            issue_gathers(i + 1, q)

        compute(p)
        issue_outs(i, p)

    @pl.loop(0, NBLK // 2 + 1)
    def _(b):
        i0 = 2 * b

        @pl.when(i0 < NBLK)
        def _():
            step(i0, 0)

        i1 = 2 * b + 1

        @pl.when(i1 < NBLK)
        def _():
            step(i1, 1)

    wait_outs((NBLK - 1) % 2)
    wait_outs(NBLK % 2)


def _sc_pass1_l1(pk, as1, ad1):
    @functools.partial(
        pl.kernel, mesh=_mesh, compiler_params=_sc_params,
        out_type=[
            jax.ShapeDtypeStruct((ET_PAD, HP), _f32),
            jax.ShapeDtypeStruct((NCORE, N_PAD, HP), _f32),
        ],
        scratch_types=[
            pltpu.VMEM_SHARED((N_PAD, HP), _f32),
            pltpu.VMEM((NBLK, 2, EB), jnp.int32),
            pltpu.VMEM((EB,), jnp.int32), pltpu.VMEM((EB,), jnp.int32),
            pltpu.VMEM((EB,), jnp.int32), pltpu.VMEM((EB,), jnp.int32),
            pltpu.VMEM((EB, HP), _f32), pltpu.VMEM((EB, HP), _f32),
            pltpu.VMEM((EB, HP), _f32), pltpu.VMEM((EB, HP), _f32),
            pltpu.VMEM((EB, HP), _f32), pltpu.VMEM((EB, HP), _f32),
        ] + [pltpu.SemaphoreType.DMA] * 8,
    )
    def k(pk_hbm, as_hbm, ad_hbm, ex_hbm, den_hbm,
          den_sh, ibt, si0, si1, di0, di1, ga0, ga1, gb0, gb1, exb0, exb1,
          ga_s0, ga_s1, gb_s0, gb_s1, sd_s0, sd_s1, se_s0, se_s1):
        c = lax.axis_index("c")
        s = lax.axis_index("s")
        wid = c * NSUB + s
        sidx = (si0, si1)
        didx = (di0, di1)
        ga = (ga0, ga1)
        gb = (gb0, gb1)
        exb = (exb0, exb1)
        ga_s = (ga_s0, ga_s1)
        gb_s = (gb_s0, gb_s1)
        sd_s = (sd_s0, sd_s1)
        se_s = (se_s0, se_s1)

        pltpu.sync_copy(pk_hbm.at[pl.ds(wid * NBLK, NBLK)], ibt)

        @pl.loop(0, EB)
        def _(r):
            exb0.at[r][...] = jnp.zeros((HP,), _f32)

        row0 = s * ROWS_PER_TILE
        _zero_rows(exb0, den_sh, row0)
        plsc.subcore_barrier()

        def issue_gathers(j, p):
            pltpu.async_copy(as_hbm.at[sidx[p]], ga[p], ga_s[p])
            pltpu.async_copy(ad_hbm.at[didx[p]], gb[p], gb_s[p])

        def wait_gathers(p):
            pltpu.make_async_copy(as_hbm.at[sidx[p]], ga[p], ga_s[p]).wait()
            pltpu.make_async_copy(ad_hbm.at[didx[p]], gb[p], gb_s[p]).wait()

        def compute(p):
            @pl.loop(0, EB)
            def _(e):
                a = ga[p].at[e][...] + gb[p].at[e][...]
                exb[p].at[e][...] = jnp.exp(jnp.maximum(a, 0.2 * a))

        def issue_outs(j, p):
            eoff = (wid * NBLK + j) * EB
            pltpu.async_copy(exb[p], den_sh.at[didx[p]], sd_s[p], add=True)
            pltpu.async_copy(exb[p], ex_hbm.at[pl.ds(eoff, EB)], se_s[p])

        def wait_outs(p):
            pltpu.make_async_copy(exb[p], den_sh.at[didx[p]], sd_s[p]).wait()
            pltpu.make_async_copy(exb[p], ex_hbm.at[pl.ds(0, EB)],
                                  se_s[p]).wait()

        _pipeline(_stage_idx(ibt, sidx, didx), issue_gathers, wait_gathers,
                  compute, issue_outs, wait_outs)

        plsc.subcore_barrier()
        pltpu.sync_copy(den_sh.at[pl.ds(row0, ROWS_PER_TILE)],
                        den_hbm.at[c].at[pl.ds(row0, ROWS_PER_TILE)])

    return k(pk, as1, ad1)


def _sc_pass2_l1(pk, ex1, inv1, xp1a, xp1b):
    DH = D1 // 2

    @functools.partial(
        pl.kernel, mesh=_mesh, compiler_params=_sc_params,
        out_type=[
            jax.ShapeDtypeStruct((NCORE, N_PAD, DH), _f32),
            jax.ShapeDtypeStruct((NCORE, N_PAD, DH), _f32),
        ],
        scratch_types=[
            pltpu.VMEM_SHARED((N_PAD, DH), _f32),
            pltpu.VMEM((NBLK, 2, EB), jnp.int32),
            pltpu.VMEM((EB,), jnp.int32), pltpu.VMEM((EB,), jnp.int32),
            pltpu.VMEM((EB,), jnp.int32), pltpu.VMEM((EB,), jnp.int32),
            pltpu.VMEM((EB, HP), _f32), pltpu.VMEM((EB, HP), _f32),
            pltpu.VMEM((EB, HP), _f32), pltpu.VMEM((EB, HP), _f32),
            pltpu.VMEM((EB, DH), _f32), pltpu.VMEM((EB, DH), _f32),
        ] + [pltpu.SemaphoreType.DMA] * 8,
    )
    def k(pk_hbm, ex_hbm, inv_hbm, xpa_hbm, xpb_hbm, outa_hbm, outb_hbm,
          out_sh, ibt, si0, si1, di0, di1, exb0, exb1, ivb0, ivb1, xsb0, xsb1,
          ge_s0, ge_s1, gi_s0, gi_s1, gx_s0, gx_s1, so_s0, so_s1):
        c = lax.axis_index("c")
        s = lax.axis_index("s")
        wid = c * NSUB + s
        sidx = (si0, si1)
        didx = (di0, di1)
        exb = (exb0, exb1)
        ivb = (ivb0, ivb1)
        xsb = (xsb0, xsb1)
        ge_s = (ge_s0, ge_s1)
        gi_s = (gi_s0, gi_s1)
        gx_s = (gx_s0, gx_s1)
        so_s = (so_s0, so_s1)
        row0 = s * ROWS_PER_TILE

        pltpu.sync_copy(pk_hbm.at[pl.ds(wid * NBLK, NBLK)], ibt)

        @pl.loop(0, EB)
        def _(r):
            row = xsb0.at[r]
            for h in range(DH // 16):
                row[pl.ds(h * 16, 16)] = jnp.zeros((16,), _f32)

        _zero_rows(xsb0, out_sh, row0)
        plsc.subcore_barrier()

        for f, (xp_hbm, o_hbm) in enumerate(((xpa_hbm, outa_hbm),
                                             (xpb_hbm, outb_hbm))):
            def issue_gathers(j, p):
                eoff = (wid * NBLK + j) * EB
                pltpu.async_copy(ex_hbm.at[pl.ds(eoff, EB)], exb[p], ge_s[p])
                pltpu.async_copy(inv_hbm.at[didx[p]], ivb[p], gi_s[p])
                pltpu.async_copy(xp_hbm.at[sidx[p]], xsb[p], gx_s[p])

            def wait_gathers(p):
                pltpu.make_async_copy(ex_hbm.at[pl.ds(0, EB)], exb[p],
                                      ge_s[p]).wait()
                pltpu.make_async_copy(inv_hbm.at[didx[p]], ivb[p],
                                      gi_s[p]).wait()
                pltpu.make_async_copy(xp_hbm.at[sidx[p]], xsb[p],
                                      gx_s[p]).wait()

            def compute(p):
                @pl.loop(0, EB)
                def _(e):
                    arow = exb[p].at[e][...] * ivb[p].at[e][...]
                    row = xsb[p].at[e]
                    for h in range(DH // 16):
                        av = jnp.full((16,), arow[f * (DH // 16) + h], _f32)
                        row[pl.ds(h * 16, 16)] = row[pl.ds(h * 16, 16)] * av

            def issue_outs(j, p):
                pltpu.async_copy(xsb[p], out_sh.at[didx[p]], so_s[p],
                                 add=True)

            def wait_outs(p):
                pltpu.make_async_copy(xsb[p], out_sh.at[didx[p]],
                                      so_s[p]).wait()

            _pipeline(_stage_idx(ibt, sidx, didx), issue_gathers,
                      wait_gathers, compute, issue_outs, wait_outs)

            plsc.subcore_barrier()
            pltpu.sync_copy(out_sh.at[pl.ds(row0, ROWS_PER_TILE)],
                            o_hbm.at[c].at[pl.ds(row0, ROWS_PER_TILE)])
            if f == 0:
                @pl.loop(0, EB)
                def _(r):
                    row = xsb0.at[r]
                    for h in range(DH // 16):
                        row[pl.ds(h * 16, 16)] = jnp.zeros((16,), _f32)

                _zero_rows(xsb0, out_sh, row0)
                plsc.subcore_barrier()

    return k(pk, ex1, inv1, xp1a, xp1b)


def _sc_pass1_l2(pk, as2, ad2):
    @functools.partial(
        pl.kernel, mesh=_mesh, compiler_params=_sc_params,
        out_type=[
            jax.ShapeDtypeStruct((ET_PAD,), _f32),
            jax.ShapeDtypeStruct((NTILE, N_PAD), _f32),
        ],
        scratch_types=[
            pltpu.VMEM((N_PAD,), _f32),
            pltpu.VMEM((N_PAD,), _f32),
            pltpu.VMEM((N_PAD,), _f32),
            pltpu.VMEM((NBLK, 2, EB), jnp.int32),
            pltpu.VMEM((EB,), _f32), pltpu.VMEM((EB,), _f32),
            pltpu.SemaphoreType.DMA, pltpu.SemaphoreType.DMA,
        ],
    )
    def k(pk_hbm, as_hbm, ad_hbm, ex_hbm, den_hbm,
          tabs, tabd, den, ibt, exb0, exb1, ssem0, ssem1):
        c = lax.axis_index("c")
        s = lax.axis_index("s")
        wid = c * NSUB + s
        exb = (exb0, exb1)
        ssem = (ssem0, ssem1)

        pltpu.sync_copy(pk_hbm.at[pl.ds(wid * NBLK, NBLK)], ibt)
        pltpu.sync_copy(as_hbm, tabs)
        pltpu.sync_copy(ad_hbm, tabd)

        @pl.loop(0, N_PAD, step=16)
        def _(i):
            den[pl.ds(i, 16)] = jnp.zeros((16,), _f32)

        def step(blk, p):
            @pl.when(blk >= 2)
            def _():
                pltpu.make_async_copy(exb[p], ex_hbm.at[pl.ds(0, EB)],
                                      ssem[p]).wait()

            sidx = ibt.at[blk, 0]
            didx = ibt.at[blk, 1]

            @pl.loop(0, EB, step=16)
            def _(e):
                s16 = sidx[pl.ds(e, 16)]
                d16 = didx[pl.ds(e, 16)]
                a = plsc.load_gather(tabs, [s16]) + plsc.load_gather(tabd,
                                                                     [d16])
                ex = jnp.exp(jnp.maximum(a, 0.2 * a))
                exb[p][pl.ds(e, 16)] = ex
                plsc.addupdate_scatter(den, [d16], ex)

            eoff = (wid * NBLK + blk) * EB
            pltpu.async_copy(exb[p], ex_hbm.at[pl.ds(eoff, EB)], ssem[p])

        @pl.loop(0, NBLK // 2 + 1)
        def _(b):
            i0 = 2 * b

            @pl.when(i0 < NBLK)
            def _():
                step(i0, 0)

            i1 = 2 * b + 1

            @pl.when(i1 < NBLK)
            def _():
                step(i1, 1)

        pltpu.make_async_copy(exb[(NBLK - 1) % 2], ex_hbm.at[pl.ds(0, EB)],
                              ssem[(NBLK - 1) % 2]).wait()
        pltpu.make_async_copy(exb[NBLK % 2], ex_hbm.at[pl.ds(0, EB)],
                              ssem[NBLK % 2]).wait()
        pltpu.sync_copy(den, den_hbm.at[wid])

    return k(pk, as2, ad2)


def _sc_pass2_l2(pk, ex2, inv2, xp2):
    @functools.partial(
        pl.kernel, mesh=_mesh, compiler_params=_sc_params,
        out_type=jax.ShapeDtypeStruct((NCORE, N_PAD, D2P), _f32),
        scratch_types=[
            pltpu.VMEM_SHARED((N_PAD, D2P), _f32),
            pltpu.VMEM((N_PAD,), _f32),
            pltpu.VMEM((NBLK, 2, EB), jnp.int32),
            pltpu.VMEM((EB,), jnp.int32), pltpu.VMEM((EB,), jnp.int32),
            pltpu.VMEM((EB,), jnp.int32), pltpu.VMEM((EB,), jnp.int32),
            pltpu.VMEM((EB,), _f32), pltpu.VMEM((EB,), _f32),
            pltpu.VMEM((EB, D2P), _f32), pltpu.VMEM((EB, D2P), _f32),
        ] + [pltpu.SemaphoreType.DMA] * 6,
    )
    def k(pk_hbm, ex_hbm, inv_hbm, xp_hbm, out_hbm,
          out_sh, tabi, ibt, si0, si1, di0, di1, exb0, exb1, xsb0, xsb1,
          ge_s0, ge_s1, gx_s0, gx_s1, so_s0, so_s1):
        c = lax.axis_index("c")
        s = lax.axis_index("s")
        wid = c * NSUB + s
        sidx = (si0, si1)
        didx = (di0, di1)
        exb = (exb0, exb1)
        xsb = (xsb0, xsb1)
        ge_s = (ge_s0, ge_s1)
        gx_s = (gx_s0, gx_s1)
        so_s = (so_s0, so_s1)

        pltpu.sync_copy(pk_hbm.at[pl.ds(wid * NBLK, NBLK)], ibt)
        pltpu.sync_copy(inv_hbm, tabi)

        @pl.loop(0, EB)
        def _(r):
            row = xsb0.at[r]
            for h in range(D2P // 16):
                row[pl.ds(h * 16, 16)] = jnp.zeros((16,), _f32)

        row0 = s * ROWS_PER_TILE
        _zero_rows(xsb0, out_sh, row0)
        plsc.subcore_barrier()

        def issue_gathers(j, p):
            eoff = (wid * NBLK + j) * EB
            pltpu.async_copy(ex_hbm.at[pl.ds(eoff, EB)], exb[p], ge_s[p])
            pltpu.async_copy(xp_hbm.at[sidx[p]], xsb[p], gx_s[p])

        def wait_gathers(p):
            pltpu.make_async_copy(ex_hbm.at[pl.ds(0, EB)], exb[p],
                                  ge_s[p]).wait()
            pltpu.make_async_copy(xp_hbm.at[sidx[p]], xsb[p],
                                  gx_s[p]).wait()

        def compute(p):
            @pl.loop(0, EB, step=16)
            def _(e):
                d16 = didx[p][pl.ds(e, 16)]
                a16 = exb[p][pl.ds(e, 16)] * plsc.load_gather(tabi, [d16])
                for j in range(16):
                    av = jnp.full((16,), a16[j], _f32)
                    row = xsb[p].at[e + j]
                    for h in range(D2P // 16):
                        row[pl.ds(h * 16, 16)] = row[pl.ds(h * 16, 16)] * av

        def issue_outs(j, p):
            pltpu.async_copy(xsb[p], out_sh.at[didx[p]], so_s[p], add=True)

        def wait_outs(p):
            pltpu.make_async_copy(xsb[p], out_sh.at[didx[p]], so_s[p]).wait()

        _pipeline(_stage_idx(ibt, sidx, didx), issue_gathers, wait_gathers,
                  compute, issue_outs, wait_outs)

        plsc.subcore_barrier()
        pltpu.sync_copy(out_sh.at[pl.ds(row0, ROWS_PER_TILE)],
                        out_hbm.at[c].at[pl.ds(row0, ROWS_PER_TILE)])

    return k(pk, ex2, inv2, xp2)


def kernel(x, edge_index, W1, att_src1, att_dst1, b1, W2, att_src2, att_dst2,
           b2):
    i32 = jnp.int32
    npad_edges = ET_PAD - E - N
    loop = jnp.arange(N, dtype=i32)
    padv = N + (jnp.arange(npad_edges, dtype=i32) % (N_PAD - N))
    src = jnp.concatenate([edge_index[0], loop, padv])
    dst = jnp.concatenate([edge_index[1], loop, padv])
    pk = jnp.stack([src.reshape(NTILE * NBLK, EB),
                    dst.reshape(NTILE * NBLK, EB)], axis=1)

    x_pad = jnp.pad(x, ((0, N_PAD - N), (0, 0)))
    a1s = jnp.pad((att_src1[:, :, None] * jnp.eye(H1, dtype=_f32)[:, None, :])
                  .reshape(D1, H1), ((0, 0), (0, HP - H1)))
    a1d = jnp.pad((att_dst1[:, :, None] * jnp.eye(H1, dtype=_f32)[:, None, :])
                  .reshape(D1, H1), ((0, 0), (0, HP - H1)))
    w2p = jnp.pad(W2, ((0, 0), (0, D2P - C2)))
    a2 = jnp.zeros((D2P, HP), _f32)
    a2 = a2.at[:C2, 0].set(att_src2[0]).at[:C2, 1].set(att_dst2[0])
    b1r = b1.reshape(1, D1)
    b2p = jnp.pad(b2, (0, D2P - C2)).reshape(1, D2P)

    xp1a, xp1b, as1, ad1 = _tc_proj1(x_pad, W1, a1s, a1d)
    ex1, den1p = _sc_pass1_l1(pk, as1, ad1)
    inv1 = _tc_inv1(den1p)
    out1pa, out1pb = _sc_pass2_l1(pk, ex1, inv1, xp1a, xp1b)

    xp2, aall = _tc_proj2(out1pa, out1pb, b1r, w2p, a2)
    as2 = aall[:, 0]
    ad2 = aall[:, 1]
    ex2, den2p = _sc_pass1_l2(pk, as2, ad2)
    inv2 = _tc_inv2(den2p.reshape(NTILE, 80, 128)).reshape(N_PAD)
    out2p = _sc_pass2_l2(pk, ex2, inv2, xp2)

    res = _tc_final(out2p, b2p)
    return res[:N, :C2]


# speedup vs baseline: 116.3850x; 1.2797x over previous
import functools

import jax
import jax.numpy as jnp
from jax import lax
from jax.experimental import pallas as pl
from jax.experimental.pallas import tpu as pltpu
from jax.experimental.pallas import tpu_sc as plsc

N = 10000
E = 320000
F = 128
H1, C1, D1 = 8, 16, 128
C2 = 40
HP = 16
D2P = 48

N_PAD = 10240
NCORE, NSUB = 2, 16
NTILE = NCORE * NSUB
ROWS_PER_TILE = N_PAD // NSUB
EB = 256
NBLK = 41
ET_PAD = NTILE * EB * NBLK

_f32 = jnp.float32
_mesh = plsc.VectorSubcoreMesh(core_axis_name="c", subcore_axis_name="s")
_sc_params = pltpu.CompilerParams(use_tc_tiling_on_sc=False,
                                  needs_layout_passes=False)


def _tc_proj1(x_pad, w1, a1s, a1d):
    def body(x_ref, w_ref, s_ref, d_ref, xpa_ref, xpb_ref, as_ref, ad_ref):
        xp = jnp.dot(x_ref[...], w_ref[...], preferred_element_type=_f32)
        xpa_ref[...] = xp[:, :D1 // 2]
        xpb_ref[...] = xp[:, D1 // 2:]
        as_ref[...] = jnp.dot(xp, s_ref[...], preferred_element_type=_f32)
        ad_ref[...] = jnp.dot(xp, d_ref[...], preferred_element_type=_f32)

    return pl.pallas_call(
        body,
        out_shape=[
            jax.ShapeDtypeStruct((N_PAD, D1 // 2), _f32),
            jax.ShapeDtypeStruct((N_PAD, D1 // 2), _f32),
            jax.ShapeDtypeStruct((N_PAD, HP), _f32),
            jax.ShapeDtypeStruct((N_PAD, HP), _f32),
        ],
    )(x_pad, w1, a1s, a1d)


def _tc_proj2(out1_pa, out1_pb, b1, w2p, a2):
    def body(pa_ref, pb_ref, b_ref, w_ref, a_ref, xp_ref, al_ref):
        h = jnp.concatenate([pa_ref[0] + pa_ref[1], pb_ref[0] + pb_ref[1]],
                            axis=1) + b_ref[...]
        xp = jnp.dot(h, w_ref[...], preferred_element_type=_f32)
        xp_ref[...] = xp
        al_ref[...] = jnp.dot(xp, a_ref[...], preferred_element_type=_f32)

    return pl.pallas_call(
        body,
        out_shape=[
            jax.ShapeDtypeStruct((N_PAD, D2P), _f32),
            jax.ShapeDtypeStruct((N_PAD, HP), _f32),
        ],
    )(out1_pa, out1_pb, b1, w2p, a2)


def _tc_inv2(den2_p):
    def body(d_ref, o_ref):
        o_ref[...] = 1.0 / (jnp.sum(d_ref[...], axis=0) + 1e-16)

    return pl.pallas_call(
        body, out_shape=jax.ShapeDtypeStruct((80, 128), _f32)
    )(den2_p)


def _tc_final(out2_p, b2p):
    def body(p_ref, b_ref, o_ref):
        o_ref[...] = p_ref[0] + p_ref[1] + b_ref[...]

    return pl.pallas_call(
        body, out_shape=jax.ShapeDtypeStruct((N_PAD, D2P), _f32)
    )(out2_p, b2p)


def _zero_rows(zbuf, dst_sh, row0):
    pltpu.sync_copy(zbuf, dst_sh.at[pl.ds(row0, EB)])
    pltpu.sync_copy(zbuf, dst_sh.at[pl.ds(row0 + EB, EB)])
    pltpu.sync_copy(zbuf.at[pl.ds(0, ROWS_PER_TILE - 2 * EB)],
                    dst_sh.at[pl.ds(row0 + 2 * EB, ROWS_PER_TILE - 2 * EB)])


def _stage_idx(ibt, sidx, didx):
    def stage(j, q):
        srow = ibt.at[j, 0]
        drow = ibt.at[j, 1]

        @plsc.parallel_loop(0, EB, 16, unroll=4)
        def _(t):
            sidx[q][pl.ds(t, 16)] = srow[pl.ds(t, 16)]
            didx[q][pl.ds(t, 16)] = drow[pl.ds(t, 16)]

    return stage


def _pipeline(stage_idx, issue_gathers, wait_gathers, compute, issue_outs,
              wait_outs):
    stage_idx(0, 0)
    issue_gathers(0, 0)

    def step(i, p):
        q = 1 - p
        wait_gathers(p)

        @pl.when(i + 1 < NBLK)
        def _():
            @pl.when(i >= 1)
            def _():
                wait_outs(q)
            stage_idx(i + 1, q)
            issue_gathers(i + 1, q)

        compute(p)
        issue_outs(i, p)

    @pl.loop(0, NBLK // 2 + 1)
    def _(b):
        i0 = 2 * b

        @pl.when(i0 < NBLK)
        def _():
            step(i0, 0)

        i1 = 2 * b + 1

        @pl.when(i1 < NBLK)
        def _():
            step(i1, 1)

    wait_outs((NBLK - 1) % 2)
    wait_outs(NBLK % 2)


def _sc_pass1_l1(pk, as1, ad1):
    @functools.partial(
        pl.kernel, mesh=_mesh, compiler_params=_sc_params,
        out_type=[
            jax.ShapeDtypeStruct((ET_PAD, HP), _f32),
            jax.ShapeDtypeStruct((NCORE, N_PAD, HP), _f32),
        ],
        scratch_types=[
            pltpu.VMEM_SHARED((N_PAD, HP), _f32),
            pltpu.VMEM((NBLK, 2, EB), jnp.int32),
            pltpu.VMEM((EB,), jnp.int32), pltpu.VMEM((EB,), jnp.int32),
            pltpu.VMEM((EB,), jnp.int32), pltpu.VMEM((EB,), jnp.int32),
            pltpu.VMEM((EB, HP), _f32), pltpu.VMEM((EB, HP), _f32),
            pltpu.VMEM((EB, HP), _f32), pltpu.VMEM((EB, HP), _f32),
            pltpu.VMEM((EB, HP), _f32), pltpu.VMEM((EB, HP), _f32),
        ] + [pltpu.SemaphoreType.DMA] * 8,
    )
    def k(pk_hbm, as_hbm, ad_hbm, ex_hbm, den_hbm,
          den_sh, ibt, si0, si1, di0, di1, ga0, ga1, gb0, gb1, exb0, exb1,
          ga_s0, ga_s1, gb_s0, gb_s1, sd_s0, sd_s1, se_s0, se_s1):
        c = lax.axis_index("c")
        s = lax.axis_index("s")
        wid = c * NSUB + s
        sidx = (si0, si1)
        didx = (di0, di1)
        ga = (ga0, ga1)
        gb = (gb0, gb1)
        exb = (exb0, exb1)
        ga_s = (ga_s0, ga_s1)
        gb_s = (gb_s0, gb_s1)
        sd_s = (sd_s0, sd_s1)
        se_s = (se_s0, se_s1)

        pltpu.sync_copy(pk_hbm.at[pl.ds(wid * NBLK, NBLK)], ibt)

        @pl.loop(0, EB)
        def _(r):
            exb0.at[r][...] = jnp.zeros((HP,), _f32)

        row0 = s * ROWS_PER_TILE
        _zero_rows(exb0, den_sh, row0)
        plsc.subcore_barrier()

        def issue_gathers(j, p):
            pltpu.async_copy(as_hbm.at[sidx[p]], ga[p], ga_s[p])
            pltpu.async_copy(ad_hbm.at[didx[p]], gb[p], gb_s[p])

        def wait_gathers(p):
            pltpu.make_async_copy(as_hbm.at[sidx[p]], ga[p], ga_s[p]).wait()
            pltpu.make_async_copy(ad_hbm.at[didx[p]], gb[p], gb_s[p]).wait()

        def compute(p):
            @plsc.parallel_loop(0, EB, unroll=4)
            def _(e):
                a = ga[p].at[e][...] + gb[p].at[e][...]
                exb[p].at[e][...] = jnp.exp(jnp.maximum(a, 0.2 * a))

        def issue_outs(j, p):
            eoff = (wid * NBLK + j) * EB
            pltpu.async_copy(exb[p], den_sh.at[didx[p]], sd_s[p], add=True)
            pltpu.async_copy(exb[p], ex_hbm.at[pl.ds(eoff, EB)], se_s[p])

        def wait_outs(p):
            pltpu.make_async_copy(exb[p], den_sh.at[didx[p]], sd_s[p]).wait()
            pltpu.make_async_copy(exb[p], ex_hbm.at[pl.ds(0, EB)],
                                  se_s[p]).wait()

        _pipeline(_stage_idx(ibt, sidx, didx), issue_gathers, wait_gathers,
                  compute, issue_outs, wait_outs)

        plsc.subcore_barrier()
        pltpu.sync_copy(den_sh.at[pl.ds(row0, ROWS_PER_TILE)],
                        den_hbm.at[c].at[pl.ds(row0, ROWS_PER_TILE)])

    return k(pk, as1, ad1)


def _sc_pass2_l1(pk, ex1, den1p, xp1a, xp1b):
    DH = D1 // 2

    @functools.partial(
        pl.kernel, mesh=_mesh, compiler_params=_sc_params,
        out_type=[
            jax.ShapeDtypeStruct((NCORE, N_PAD, DH), _f32),
            jax.ShapeDtypeStruct((NCORE, N_PAD, DH), _f32),
        ],
        scratch_types=[
            pltpu.VMEM_SHARED((N_PAD, DH), _f32),
            pltpu.VMEM_SHARED((N_PAD, HP), _f32),
            pltpu.VMEM((NBLK, 2, EB), jnp.int32),
            pltpu.VMEM((EB,), jnp.int32), pltpu.VMEM((EB,), jnp.int32),
            pltpu.VMEM((EB,), jnp.int32), pltpu.VMEM((EB,), jnp.int32),
            pltpu.VMEM((EB, HP), _f32), pltpu.VMEM((EB, HP), _f32),
            pltpu.VMEM((EB, HP), _f32), pltpu.VMEM((EB, HP), _f32),
            pltpu.VMEM((EB, DH), _f32), pltpu.VMEM((EB, DH), _f32),
        ] + [pltpu.SemaphoreType.DMA] * 8,
    )
    def k(pk_hbm, ex_hbm, den_hbm, xpa_hbm, xpb_hbm, outa_hbm, outb_hbm,
          out_sh, inv_sh, ibt, si0, si1, di0, di1, exb0, exb1, ivb0, ivb1,
          xsb0, xsb1, ge_s0, ge_s1, gi_s0, gi_s1, gx_s0, gx_s1, so_s0, so_s1):
        c = lax.axis_index("c")
        s = lax.axis_index("s")
        wid = c * NSUB + s
        sidx = (si0, si1)
        didx = (di0, di1)
        exb = (exb0, exb1)
        ivb = (ivb0, ivb1)
        xsb = (xsb0, xsb1)
        ge_s = (ge_s0, ge_s1)
        gi_s = (gi_s0, gi_s1)
        gx_s = (gx_s0, gx_s1)
        so_s = (so_s0, so_s1)
        row0 = s * ROWS_PER_TILE

        pltpu.sync_copy(pk_hbm.at[pl.ds(wid * NBLK, NBLK)], ibt)

        for ck in range(ROWS_PER_TILE // 128):
            r0 = row0 + ck * 128
            pltpu.sync_copy(den_hbm.at[0].at[pl.ds(r0, 128)],
                            exb0.at[pl.ds(0, 128)])
            pltpu.sync_copy(den_hbm.at[1].at[pl.ds(r0, 128)],
                            exb0.at[pl.ds(128, 128)])

            @plsc.parallel_loop(0, 128, unroll=4)
            def _(r):
                d = exb0.at[r][...] + exb0.at[128 + r][...]
                ivb0.at[r][...] = 1.0 / (d + 1e-16)

            pltpu.sync_copy(ivb0.at[pl.ds(0, 128)], inv_sh.at[pl.ds(r0, 128)])

        @pl.loop(0, EB)
        def _(r):
            row = xsb0.at[r]
            for h in range(DH // 16):
                row[pl.ds(h * 16, 16)] = jnp.zeros((16,), _f32)

        _zero_rows(xsb0, out_sh, row0)
        plsc.subcore_barrier()

        for f, (xp_hbm, o_hbm) in enumerate(((xpa_hbm, outa_hbm),
                                             (xpb_hbm, outb_hbm))):
            def issue_gathers(j, p):
                eoff = (wid * NBLK + j) * EB
                pltpu.async_copy(ex_hbm.at[pl.ds(eoff, EB)], exb[p], ge_s[p])
                pltpu.async_copy(inv_sh.at[didx[p]], ivb[p], gi_s[p])
                pltpu.async_copy(xp_hbm.at[sidx[p]], xsb[p], gx_s[p])

            def wait_gathers(p):
                pltpu.make_async_copy(ex_hbm.at[pl.ds(0, EB)], exb[p],
                                      ge_s[p]).wait()
                pltpu.make_async_copy(inv_sh.at[didx[p]], ivb[p],
                                      gi_s[p]).wait()
                pltpu.make_async_copy(xp_hbm.at[sidx[p]], xsb[p],
                                      gx_s[p]).wait()

            def compute(p):
                @plsc.parallel_loop(0, EB, unroll=2)
                def _(e):
                    arow = exb[p].at[e][...] * ivb[p].at[e][...]
                    row = xsb[p].at[e]
                    for h in range(DH // 16):
                        av = jnp.full((16,), arow[f * (DH // 16) + h], _f32)
                        row[pl.ds(h * 16, 16)] = row[pl.ds(h * 16, 16)] * av

            def issue_outs(j, p):
                pltpu.async_copy(xsb[p], out_sh.at[didx[p]], so_s[p],
                                 add=True)

            def wait_outs(p):
                pltpu.make_async_copy(xsb[p], out_sh.at[didx[p]],
                                      so_s[p]).wait()

            _pipeline(_stage_idx(ibt, sidx, didx), issue_gathers,
                      wait_gathers, compute, issue_outs, wait_outs)

            plsc.subcore_barrier()
            pltpu.sync_copy(out_sh.at[pl.ds(row0, ROWS_PER_TILE)],
                            o_hbm.at[c].at[pl.ds(row0, ROWS_PER_TILE)])
            if f == 0:
                @pl.loop(0, EB)
                def _(r):
                    row = xsb0.at[r]
                    for h in range(DH // 16):
                        row[pl.ds(h * 16, 16)] = jnp.zeros((16,), _f32)

                _zero_rows(xsb0, out_sh, row0)
                plsc.subcore_barrier()

    return k(pk, ex1, den1p, xp1a, xp1b)


def _sc_pass1_l2(pk, as2, ad2):
    @functools.partial(
        pl.kernel, mesh=_mesh, compiler_params=_sc_params,
        out_type=[
            jax.ShapeDtypeStruct((ET_PAD,), _f32),
            jax.ShapeDtypeStruct((NTILE, N_PAD), _f32),
        ],
        scratch_types=[
            pltpu.VMEM((N_PAD,), _f32),
            pltpu.VMEM((N_PAD,), _f32),
            pltpu.VMEM((N_PAD,), _f32),
            pltpu.VMEM((NBLK, 2, EB), jnp.int32),
            pltpu.VMEM((EB,), _f32), pltpu.VMEM((EB,), _f32),
            pltpu.SemaphoreType.DMA, pltpu.SemaphoreType.DMA,
        ],
    )
    def k(pk_hbm, as_hbm, ad_hbm, ex_hbm, den_hbm,
          tabs, tabd, den, ibt, exb0, exb1, ssem0, ssem1):
        c = lax.axis_index("c")
        s = lax.axis_index("s")
        wid = c * NSUB + s
        exb = (exb0, exb1)
        ssem = (ssem0, ssem1)

        pltpu.sync_copy(pk_hbm.at[pl.ds(wid * NBLK, NBLK)], ibt)
        pltpu.sync_copy(as_hbm, tabs)
        pltpu.sync_copy(ad_hbm, tabd)

        @pl.loop(0, N_PAD, step=16)
        def _(i):
            den[pl.ds(i, 16)] = jnp.zeros((16,), _f32)

        def step(blk, p):
            @pl.when(blk >= 2)
            def _():
                pltpu.make_async_copy(exb[p], ex_hbm.at[pl.ds(0, EB)],
                                      ssem[p]).wait()

            sidx = ibt.at[blk, 0]
            didx = ibt.at[blk, 1]

            @pl.loop(0, EB, step=16)
            def _(e):
                s16 = sidx[pl.ds(e, 16)]
                d16 = didx[pl.ds(e, 16)]
                a = plsc.load_gather(tabs, [s16]) + plsc.load_gather(tabd,
                                                                     [d16])
                ex = jnp.exp(jnp.maximum(a, 0.2 * a))
                exb[p][pl.ds(e, 16)] = ex
                plsc.addupdate_scatter(den, [d16], ex)

            eoff = (wid * NBLK + blk) * EB
            pltpu.async_copy(exb[p], ex_hbm.at[pl.ds(eoff, EB)], ssem[p])

        @pl.loop(0, NBLK // 2 + 1)
        def _(b):
            i0 = 2 * b

            @pl.when(i0 < NBLK)
            def _():
                step(i0, 0)

            i1 = 2 * b + 1

            @pl.when(i1 < NBLK)
            def _():
                step(i1, 1)

        pltpu.make_async_copy(exb[(NBLK - 1) % 2], ex_hbm.at[pl.ds(0, EB)],
                              ssem[(NBLK - 1) % 2]).wait()
        pltpu.make_async_copy(exb[NBLK % 2], ex_hbm.at[pl.ds(0, EB)],
                              ssem[NBLK % 2]).wait()
        pltpu.sync_copy(den, den_hbm.at[wid])

    return k(pk, as2, ad2)


def _sc_pass2_l2(pk, ex2, inv2, xp2):
    @functools.partial(
        pl.kernel, mesh=_mesh, compiler_params=_sc_params,
        out_type=jax.ShapeDtypeStruct((NCORE, N_PAD, D2P), _f32),
        scratch_types=[
            pltpu.VMEM_SHARED((N_PAD, D2P), _f32),
            pltpu.VMEM((N_PAD,), _f32),
            pltpu.VMEM((NBLK, 2, EB), jnp.int32),
            pltpu.VMEM((EB,), jnp.int32), pltpu.VMEM((EB,), jnp.int32),
            pltpu.VMEM((EB,), jnp.int32), pltpu.VMEM((EB,), jnp.int32),
            pltpu.VMEM((EB,), _f32), pltpu.VMEM((EB,), _f32),
            pltpu.VMEM((EB, D2P), _f32), pltpu.VMEM((EB, D2P), _f32),
        ] + [pltpu.SemaphoreType.DMA] * 6,
    )
    def k(pk_hbm, ex_hbm, inv_hbm, xp_hbm, out_hbm,
          out_sh, tabi, ibt, si0, si1, di0, di1, exb0, exb1, xsb0, xsb1,
          ge_s0, ge_s1, gx_s0, gx_s1, so_s0, so_s1):
        c = lax.axis_index("c")
        s = lax.axis_index("s")
        wid = c * NSUB + s
        sidx = (si0, si1)
        didx = (di0, di1)
        exb = (exb0, exb1)
        xsb = (xsb0, xsb1)
        ge_s = (ge_s0, ge_s1)
        gx_s = (gx_s0, gx_s1)
        so_s = (so_s0, so_s1)

        pltpu.sync_copy(pk_hbm.at[pl.ds(wid * NBLK, NBLK)], ibt)
        pltpu.sync_copy(inv_hbm, tabi)

        @pl.loop(0, EB)
        def _(r):
            row = xsb0.at[r]
            for h in range(D2P // 16):
                row[pl.ds(h * 16, 16)] = jnp.zeros((16,), _f32)

        row0 = s * ROWS_PER_TILE
        _zero_rows(xsb0, out_sh, row0)
        plsc.subcore_barrier()

        def issue_gathers(j, p):
            eoff = (wid * NBLK + j) * EB
            pltpu.async_copy(ex_hbm.at[pl.ds(eoff, EB)], exb[p], ge_s[p])
            pltpu.async_copy(xp_hbm.at[sidx[p]], xsb[p], gx_s[p])

        def wait_gathers(p):
            pltpu.make_async_copy(ex_hbm.at[pl.ds(0, EB)], exb[p],
                                  ge_s[p]).wait()
            pltpu.make_async_copy(xp_hbm.at[sidx[p]], xsb[p],
                                  gx_s[p]).wait()

        def compute(p):
            @plsc.parallel_loop(0, EB, 16, unroll=2)
            def _(e):
                d16 = didx[p][pl.ds(e, 16)]
                a16 = exb[p][pl.ds(e, 16)] * plsc.load_gather(tabi, [d16])
                for j in range(16):
                    av = jnp.full((16,), a16[j], _f32)
                    row = xsb[p].at[e + j]
                    for h in range(D2P // 16):
                        row[pl.ds(h * 16, 16)] = row[pl.ds(h * 16, 16)] * av

        def issue_outs(j, p):
            pltpu.async_copy(xsb[p], out_sh.at[didx[p]], so_s[p], add=True)

        def wait_outs(p):
            pltpu.make_async_copy(xsb[p], out_sh.at[didx[p]], so_s[p]).wait()

        _pipeline(_stage_idx(ibt, sidx, didx), issue_gathers, wait_gathers,
                  compute, issue_outs, wait_outs)

        plsc.subcore_barrier()
        pltpu.sync_copy(out_sh.at[pl.ds(row0, ROWS_PER_TILE)],
                        out_hbm.at[c].at[pl.ds(row0, ROWS_PER_TILE)])

    return k(pk, ex2, inv2, xp2)


def kernel(x, edge_index, W1, att_src1, att_dst1, b1, W2, att_src2, att_dst2,
           b2):
    i32 = jnp.int32
    npad_edges = ET_PAD - E - N
    loop = jnp.arange(N, dtype=i32)
    padv = N + (jnp.arange(npad_edges, dtype=i32) % (N_PAD - N))
    src = jnp.concatenate([edge_index[0], loop, padv])
    dst = jnp.concatenate([edge_index[1], loop, padv])
    pk = jnp.stack([src.reshape(NTILE * NBLK, EB),
                    dst.reshape(NTILE * NBLK, EB)], axis=1)

    x_pad = jnp.pad(x, ((0, N_PAD - N), (0, 0)))
    a1s = jnp.pad((att_src1[:, :, None] * jnp.eye(H1, dtype=_f32)[:, None, :])
                  .reshape(D1, H1), ((0, 0), (0, HP - H1)))
    a1d = jnp.pad((att_dst1[:, :, None] * jnp.eye(H1, dtype=_f32)[:, None, :])
                  .reshape(D1, H1), ((0, 0), (0, HP - H1)))
    w2p = jnp.pad(W2, ((0, 0), (0, D2P - C2)))
    a2 = jnp.zeros((D2P, HP), _f32)
    a2 = a2.at[:C2, 0].set(att_src2[0]).at[:C2, 1].set(att_dst2[0])
    b1r = b1.reshape(1, D1)
    b2p = jnp.pad(b2, (0, D2P - C2)).reshape(1, D2P)

    xp1a, xp1b, as1, ad1 = _tc_proj1(x_pad, W1, a1s, a1d)
    ex1, den1p = _sc_pass1_l1(pk, as1, ad1)
    out1pa, out1pb = _sc_pass2_l1(pk, ex1, den1p, xp1a, xp1b)

    xp2, aall = _tc_proj2(out1pa, out1pb, b1r, w2p, a2)
    as2 = aall[:, 0]
    ad2 = aall[:, 1]
    ex2, den2p = _sc_pass1_l2(pk, as2, ad2)
    inv2 = _tc_inv2(den2p.reshape(NTILE, 80, 128)).reshape(N_PAD)
    out2p = _sc_pass2_l2(pk, ex2, inv2, xp2)

    res = _tc_final(out2p, b2p)
    return res[:N, :C2]


# speedup vs baseline: 116.9011x; 1.0044x over previous
import functools

import jax
import jax.numpy as jnp
from jax import lax
from jax.experimental import pallas as pl
from jax.experimental.pallas import tpu as pltpu
from jax.experimental.pallas import tpu_sc as plsc

N = 10000
E = 320000
F = 128
H1, C1, D1 = 8, 16, 128
C2 = 40
HP = 16
D2P = 48

N_PAD = 10240
NCORE, NSUB = 2, 16
NTILE = NCORE * NSUB
ROWS_PER_TILE = N_PAD // NSUB
EB = 256
NBLK = 41
ET_PAD = NTILE * EB * NBLK

_f32 = jnp.float32
_mesh = plsc.VectorSubcoreMesh(core_axis_name="c", subcore_axis_name="s")
_sc_params = pltpu.CompilerParams(use_tc_tiling_on_sc=False,
                                  needs_layout_passes=False)


def _tc_proj1(x_pad, w1, a1s, a1d):
    def body(x_ref, w_ref, s_ref, d_ref, xpa_ref, xpb_ref, as_ref, ad_ref):
        xp = jnp.dot(x_ref[...], w_ref[...], preferred_element_type=_f32)
        xpa_ref[...] = xp[:, :D1 // 2]
        xpb_ref[...] = xp[:, D1 // 2:]
        as_ref[...] = jnp.dot(xp, s_ref[...], preferred_element_type=_f32)
        ad_ref[...] = jnp.dot(xp, d_ref[...], preferred_element_type=_f32)

    return pl.pallas_call(
        body,
        out_shape=[
            jax.ShapeDtypeStruct((N_PAD, D1 // 2), _f32),
            jax.ShapeDtypeStruct((N_PAD, D1 // 2), _f32),
            jax.ShapeDtypeStruct((N_PAD, HP), _f32),
            jax.ShapeDtypeStruct((N_PAD, HP), _f32),
        ],
    )(x_pad, w1, a1s, a1d)


def _tc_proj2(out1_pa, out1_pb, b1, w2p, a2):
    def body(pa_ref, pb_ref, b_ref, w_ref, a_ref, xp_ref, al_ref):
        h = jnp.concatenate([pa_ref[0] + pa_ref[1], pb_ref[0] + pb_ref[1]],
                            axis=1) + b_ref[...]
        xp = jnp.dot(h, w_ref[...], preferred_element_type=_f32)
        xp_ref[...] = xp
        al_ref[...] = jnp.dot(xp, a_ref[...], preferred_element_type=_f32)

    return pl.pallas_call(
        body,
        out_shape=[
            jax.ShapeDtypeStruct((N_PAD, D2P), _f32),
            jax.ShapeDtypeStruct((N_PAD, HP), _f32),
        ],
    )(out1_pa, out1_pb, b1, w2p, a2)


def _tc_final(out2_p, b2p):
    def body(p_ref, b_ref, o_ref):
        o_ref[...] = p_ref[0] + p_ref[1] + b_ref[...]

    return pl.pallas_call(
        body, out_shape=jax.ShapeDtypeStruct((N_PAD, D2P), _f32)
    )(out2_p, b2p)


def _zero_rows(zbuf, dst_sh, row0):
    pltpu.sync_copy(zbuf, dst_sh.at[pl.ds(row0, EB)])
    pltpu.sync_copy(zbuf, dst_sh.at[pl.ds(row0 + EB, EB)])
    pltpu.sync_copy(zbuf.at[pl.ds(0, ROWS_PER_TILE - 2 * EB)],
                    dst_sh.at[pl.ds(row0 + 2 * EB, ROWS_PER_TILE - 2 * EB)])


def _stage_idx(ibt, sidx, didx):
    def stage(j, q):
        srow = ibt.at[j, 0]
        drow = ibt.at[j, 1]

        @plsc.parallel_loop(0, EB, 16, unroll=4)
        def _(t):
            sidx[q][pl.ds(t, 16)] = srow[pl.ds(t, 16)]
            didx[q][pl.ds(t, 16)] = drow[pl.ds(t, 16)]

    return stage


def _pipeline(stage_idx, issue_gathers, wait_gathers, compute, issue_outs,
              wait_outs):
    stage_idx(0, 0)
    issue_gathers(0, 0)

    def step(i, p):
        q = 1 - p
        wait_gathers(p)

        @pl.when(i + 1 < NBLK)
        def _():
            @pl.when(i >= 1)
            def _():
                wait_outs(q)
            stage_idx(i + 1, q)
            issue_gathers(i + 1, q)

        compute(p)
        issue_outs(i, p)

    @pl.loop(0, NBLK // 2 + 1)
    def _(b):
        i0 = 2 * b

        @pl.when(i0 < NBLK)
        def _():
            step(i0, 0)

        i1 = 2 * b + 1

        @pl.when(i1 < NBLK)
        def _():
            step(i1, 1)

    wait_outs((NBLK - 1) % 2)
    wait_outs(NBLK % 2)


def _sc_pass1_l1(pk, as1, ad1):
    @functools.partial(
        pl.kernel, mesh=_mesh, compiler_params=_sc_params,
        out_type=[
            jax.ShapeDtypeStruct((ET_PAD, HP), _f32),
            jax.ShapeDtypeStruct((NCORE, N_PAD, HP), _f32),
        ],
        scratch_types=[
            pltpu.VMEM_SHARED((N_PAD, HP), _f32),
            pltpu.VMEM((NBLK, 2, EB), jnp.int32),
            pltpu.VMEM((EB,), jnp.int32), pltpu.VMEM((EB,), jnp.int32),
            pltpu.VMEM((EB,), jnp.int32), pltpu.VMEM((EB,), jnp.int32),
            pltpu.VMEM((EB, HP), _f32), pltpu.VMEM((EB, HP), _f32),
            pltpu.VMEM((EB, HP), _f32), pltpu.VMEM((EB, HP), _f32),
            pltpu.VMEM((EB, HP), _f32), pltpu.VMEM((EB, HP), _f32),
        ] + [pltpu.SemaphoreType.DMA] * 8,
    )
    def k(pk_hbm, as_hbm, ad_hbm, ex_hbm, den_hbm,
          den_sh, ibt, si0, si1, di0, di1, ga0, ga1, gb0, gb1, exb0, exb1,
          ga_s0, ga_s1, gb_s0, gb_s1, sd_s0, sd_s1, se_s0, se_s1):
        c = lax.axis_index("c")
        s = lax.axis_index("s")
        wid = c * NSUB + s
        sidx = (si0, si1)
        didx = (di0, di1)
        ga = (ga0, ga1)
        gb = (gb0, gb1)
        exb = (exb0, exb1)
        ga_s = (ga_s0, ga_s1)
        gb_s = (gb_s0, gb_s1)
        sd_s = (sd_s0, sd_s1)
        se_s = (se_s0, se_s1)

        pltpu.sync_copy(pk_hbm.at[pl.ds(wid * NBLK, NBLK)], ibt)

        @pl.loop(0, EB)
        def _(r):
            exb0.at[r][...] = jnp.zeros((HP,), _f32)

        row0 = s * ROWS_PER_TILE
        _zero_rows(exb0, den_sh, row0)
        plsc.subcore_barrier()

        def issue_gathers(j, p):
            pltpu.async_copy(as_hbm.at[sidx[p]], ga[p], ga_s[p])
            pltpu.async_copy(ad_hbm.at[didx[p]], gb[p], gb_s[p])

        def wait_gathers(p):
            pltpu.make_async_copy(as_hbm.at[sidx[p]], ga[p], ga_s[p]).wait()
            pltpu.make_async_copy(ad_hbm.at[didx[p]], gb[p], gb_s[p]).wait()

        def compute(p):
            @plsc.parallel_loop(0, EB, unroll=4)
            def _(e):
                a = ga[p].at[e][...] + gb[p].at[e][...]
                exb[p].at[e][...] = jnp.exp(jnp.maximum(a, 0.2 * a))

        def issue_outs(j, p):
            eoff = (wid * NBLK + j) * EB
            pltpu.async_copy(exb[p], den_sh.at[didx[p]], sd_s[p], add=True)
            pltpu.async_copy(exb[p], ex_hbm.at[pl.ds(eoff, EB)], se_s[p])

        def wait_outs(p):
            pltpu.make_async_copy(exb[p], den_sh.at[didx[p]], sd_s[p]).wait()
            pltpu.make_async_copy(exb[p], ex_hbm.at[pl.ds(0, EB)],
                                  se_s[p]).wait()

        _pipeline(_stage_idx(ibt, sidx, didx), issue_gathers, wait_gathers,
                  compute, issue_outs, wait_outs)

        plsc.subcore_barrier()
        pltpu.sync_copy(den_sh.at[pl.ds(row0, ROWS_PER_TILE)],
                        den_hbm.at[c].at[pl.ds(row0, ROWS_PER_TILE)])

    return k(pk, as1, ad1)


def _sc_pass2_l1(pk, ex1, den1p, xp1a, xp1b):
    DH = D1 // 2

    @functools.partial(
        pl.kernel, mesh=_mesh, compiler_params=_sc_params,
        out_type=[
            jax.ShapeDtypeStruct((NCORE, N_PAD, DH), _f32),
            jax.ShapeDtypeStruct((NCORE, N_PAD, DH), _f32),
        ],
        scratch_types=[
            pltpu.VMEM_SHARED((N_PAD, DH), _f32),
            pltpu.VMEM_SHARED((N_PAD, HP), _f32),
            pltpu.VMEM((NBLK, 2, EB), jnp.int32),
            pltpu.VMEM((EB,), jnp.int32), pltpu.VMEM((EB,), jnp.int32),
            pltpu.VMEM((EB,), jnp.int32), pltpu.VMEM((EB,), jnp.int32),
            pltpu.VMEM((EB, HP), _f32), pltpu.VMEM((EB, HP), _f32),
            pltpu.VMEM((EB, HP), _f32), pltpu.VMEM((EB, HP), _f32),
            pltpu.VMEM((EB, DH), _f32), pltpu.VMEM((EB, DH), _f32),
        ] + [pltpu.SemaphoreType.DMA] * 8,
    )
    def k(pk_hbm, ex_hbm, den_hbm, xpa_hbm, xpb_hbm, outa_hbm, outb_hbm,
          out_sh, inv_sh, ibt, si0, si1, di0, di1, exb0, exb1, ivb0, ivb1,
          xsb0, xsb1, ge_s0, ge_s1, gi_s0, gi_s1, gx_s0, gx_s1, so_s0, so_s1):
        c = lax.axis_index("c")
        s = lax.axis_index("s")
        wid = c * NSUB + s
        sidx = (si0, si1)
        didx = (di0, di1)
        exb = (exb0, exb1)
        ivb = (ivb0, ivb1)
        xsb = (xsb0, xsb1)
        ge_s = (ge_s0, ge_s1)
        gi_s = (gi_s0, gi_s1)
        gx_s = (gx_s0, gx_s1)
        so_s = (so_s0, so_s1)
        row0 = s * ROWS_PER_TILE

        pltpu.sync_copy(pk_hbm.at[pl.ds(wid * NBLK, NBLK)], ibt)

        for ck in range(ROWS_PER_TILE // 128):
            r0 = row0 + ck * 128
            pltpu.sync_copy(den_hbm.at[0].at[pl.ds(r0, 128)],
                            exb0.at[pl.ds(0, 128)])
            pltpu.sync_copy(den_hbm.at[1].at[pl.ds(r0, 128)],
                            exb0.at[pl.ds(128, 128)])

            @plsc.parallel_loop(0, 128, unroll=4)
            def _(r):
                d = exb0.at[r][...] + exb0.at[128 + r][...]
                ivb0.at[r][...] = 1.0 / (d + 1e-16)

            pltpu.sync_copy(ivb0.at[pl.ds(0, 128)], inv_sh.at[pl.ds(r0, 128)])

        @pl.loop(0, EB)
        def _(r):
            row = xsb0.at[r]
            for h in range(DH // 16):
                row[pl.ds(h * 16, 16)] = jnp.zeros((16,), _f32)

        _zero_rows(xsb0, out_sh, row0)
        plsc.subcore_barrier()

        for f, (xp_hbm, o_hbm) in enumerate(((xpa_hbm, outa_hbm),
                                             (xpb_hbm, outb_hbm))):
            def issue_gathers(j, p):
                eoff = (wid * NBLK + j) * EB
                pltpu.async_copy(ex_hbm.at[pl.ds(eoff, EB)], exb[p], ge_s[p])
                pltpu.async_copy(inv_sh.at[didx[p]], ivb[p], gi_s[p])
                pltpu.async_copy(xp_hbm.at[sidx[p]], xsb[p], gx_s[p])

            def wait_gathers(p):
                pltpu.make_async_copy(ex_hbm.at[pl.ds(0, EB)], exb[p],
                                      ge_s[p]).wait()
                pltpu.make_async_copy(inv_sh.at[didx[p]], ivb[p],
                                      gi_s[p]).wait()
                pltpu.make_async_copy(xp_hbm.at[sidx[p]], xsb[p],
                                      gx_s[p]).wait()

            def compute(p):
                @plsc.parallel_loop(0, EB, unroll=4)
                def _(e):
                    arow = exb[p].at[e][...] * ivb[p].at[e][...]
                    row = xsb[p].at[e]
                    for h in range(DH // 16):
                        av = jnp.full((16,), arow[f * (DH // 16) + h], _f32)
                        row[pl.ds(h * 16, 16)] = row[pl.ds(h * 16, 16)] * av

            def issue_outs(j, p):
                pltpu.async_copy(xsb[p], out_sh.at[didx[p]], so_s[p],
                                 add=True)

            def wait_outs(p):
                pltpu.make_async_copy(xsb[p], out_sh.at[didx[p]],
                                      so_s[p]).wait()

            _pipeline(_stage_idx(ibt, sidx, didx), issue_gathers,
                      wait_gathers, compute, issue_outs, wait_outs)

            plsc.subcore_barrier()
            pltpu.sync_copy(out_sh.at[pl.ds(row0, ROWS_PER_TILE)],
                            o_hbm.at[c].at[pl.ds(row0, ROWS_PER_TILE)])
            if f == 0:
                @pl.loop(0, EB)
                def _(r):
                    row = xsb0.at[r]
                    for h in range(DH // 16):
                        row[pl.ds(h * 16, 16)] = jnp.zeros((16,), _f32)

                _zero_rows(xsb0, out_sh, row0)
                plsc.subcore_barrier()

    return k(pk, ex1, den1p, xp1a, xp1b)


def _sc_pass1_l2(pk, as2, ad2):
    NR = N_PAD // 16

    @functools.partial(
        pl.kernel, mesh=_mesh, compiler_params=_sc_params,
        out_type=[
            jax.ShapeDtypeStruct((ET_PAD,), _f32),
            jax.ShapeDtypeStruct((NCORE, NR, 16), _f32),
        ],
        scratch_types=[
            pltpu.VMEM_SHARED((NR, 16), _f32),
            pltpu.VMEM((N_PAD,), _f32),
            pltpu.VMEM((N_PAD,), _f32),
            pltpu.VMEM((NR, 16), _f32),
            pltpu.VMEM((NR,), jnp.int32),
            pltpu.VMEM((NBLK, 2, EB), jnp.int32),
            pltpu.VMEM((EB,), _f32), pltpu.VMEM((EB,), _f32),
            pltpu.SemaphoreType.DMA, pltpu.SemaphoreType.DMA,
        ],
    )
    def k(pk_hbm, as_hbm, ad_hbm, ex_hbm, den_hbm,
          den_sh, tabs, tabd, den, idr, ibt, exb0, exb1, ssem0, ssem1):
        c = lax.axis_index("c")
        s = lax.axis_index("s")
        wid = c * NSUB + s
        exb = (exb0, exb1)
        ssem = (ssem0, ssem1)

        pltpu.sync_copy(pk_hbm.at[pl.ds(wid * NBLK, NBLK)], ibt)
        pltpu.sync_copy(as_hbm, tabs)
        pltpu.sync_copy(ad_hbm, tabd)

        @pl.loop(0, NR, step=16)
        def _(i):
            z = jnp.zeros((16,), _f32)
            for t in range(16):
                den.at[i + t][...] = z
            idr[pl.ds(i, 16)] = lax.iota(jnp.int32, 16) + i

        rws = NR // NSUB
        pltpu.sync_copy(den.at[pl.ds(0, rws)], den_sh.at[pl.ds(s * rws, rws)])
        plsc.subcore_barrier()

        def step(blk, p):
            @pl.when(blk >= 2)
            def _():
                pltpu.make_async_copy(exb[p], ex_hbm.at[pl.ds(0, EB)],
                                      ssem[p]).wait()

            sidx = ibt.at[blk, 0]
            didx = ibt.at[blk, 1]

            @pl.loop(0, EB, step=16)
            def _(e):
                s16 = sidx[pl.ds(e, 16)]
                d16 = didx[pl.ds(e, 16)]
                a = plsc.load_gather(tabs, [s16]) + plsc.load_gather(tabd,
                                                                     [d16])
                ex = jnp.exp(jnp.maximum(a, 0.2 * a))
                exb[p][pl.ds(e, 16)] = ex
                plsc.addupdate_scatter(
                    den, [lax.shift_right_logical(d16, 4),
                          lax.bitwise_and(d16, 15)], ex)

            eoff = (wid * NBLK + blk) * EB
            pltpu.async_copy(exb[p], ex_hbm.at[pl.ds(eoff, EB)], ssem[p])

        @pl.loop(0, NBLK // 2 + 1)
        def _(b):
            i0 = 2 * b

            @pl.when(i0 < NBLK)
            def _():
                step(i0, 0)

            i1 = 2 * b + 1

            @pl.when(i1 < NBLK)
            def _():
                step(i1, 1)

        pltpu.make_async_copy(exb[(NBLK - 1) % 2], ex_hbm.at[pl.ds(0, EB)],
                              ssem[(NBLK - 1) % 2]).wait()
        pltpu.make_async_copy(exb[NBLK % 2], ex_hbm.at[pl.ds(0, EB)],
                              ssem[NBLK % 2]).wait()
        pltpu.sync_copy(den, den_sh.at[idr], add=True)
        plsc.subcore_barrier()
        pltpu.sync_copy(den_sh.at[pl.ds(s * rws, rws)],
                        den_hbm.at[c].at[pl.ds(s * rws, rws)])

    return k(pk, as2, ad2)


def _sc_pass2_l2(pk, ex2, den2p, xp2):
    NR = N_PAD // 16

    @functools.partial(
        pl.kernel, mesh=_mesh, compiler_params=_sc_params,
        out_type=jax.ShapeDtypeStruct((NCORE, N_PAD, D2P), _f32),
        scratch_types=[
            pltpu.VMEM_SHARED((N_PAD, D2P), _f32),
            pltpu.VMEM_SHARED((NR, 16), _f32),
            pltpu.VMEM((NR, 16), _f32),
            pltpu.VMEM((NR // NSUB, 16), _f32),
            pltpu.VMEM((NR // NSUB, 16), _f32),
            pltpu.VMEM((NBLK, 2, EB), jnp.int32),
            pltpu.VMEM((EB,), jnp.int32), pltpu.VMEM((EB,), jnp.int32),
            pltpu.VMEM((EB,), jnp.int32), pltpu.VMEM((EB,), jnp.int32),
            pltpu.VMEM((EB,), _f32), pltpu.VMEM((EB,), _f32),
            pltpu.VMEM((EB, D2P), _f32), pltpu.VMEM((EB, D2P), _f32),
        ] + [pltpu.SemaphoreType.DMA] * 6,
    )
    def k(pk_hbm, ex_hbm, den_hbm, xp_hbm, out_hbm,
          out_sh, inv_sh, tabi, t0, t1, ibt, si0, si1, di0, di1,
          exb0, exb1, xsb0, xsb1,
          ge_s0, ge_s1, gx_s0, gx_s1, so_s0, so_s1):
        c = lax.axis_index("c")
        s = lax.axis_index("s")
        wid = c * NSUB + s
        sidx = (si0, si1)
        didx = (di0, di1)
        exb = (exb0, exb1)
        xsb = (xsb0, xsb1)
        ge_s = (ge_s0, ge_s1)
        gx_s = (gx_s0, gx_s1)
        so_s = (so_s0, so_s1)
        rws = NR // NSUB

        pltpu.sync_copy(pk_hbm.at[pl.ds(wid * NBLK, NBLK)], ibt)

        pltpu.sync_copy(den_hbm.at[0].at[pl.ds(s * rws, rws)], t0)
        pltpu.sync_copy(den_hbm.at[1].at[pl.ds(s * rws, rws)], t1)

        @plsc.parallel_loop(0, rws, unroll=4)
        def _(r):
            d = t0.at[r][...] + t1.at[r][...]
            t0.at[r][...] = 1.0 / (d + 1e-16)

        pltpu.sync_copy(t0, inv_sh.at[pl.ds(s * rws, rws)])
        plsc.subcore_barrier()
        pltpu.sync_copy(inv_sh, tabi)

        @pl.loop(0, EB)
        def _(r):
            row = xsb0.at[r]
            for h in range(D2P // 16):
                row[pl.ds(h * 16, 16)] = jnp.zeros((16,), _f32)

        row0 = s * ROWS_PER_TILE
        _zero_rows(xsb0, out_sh, row0)
        plsc.subcore_barrier()

        def issue_gathers(j, p):
            eoff = (wid * NBLK + j) * EB
            pltpu.async_copy(ex_hbm.at[pl.ds(eoff, EB)], exb[p], ge_s[p])
            pltpu.async_copy(xp_hbm.at[sidx[p]], xsb[p], gx_s[p])

        def wait_gathers(p):
            pltpu.make_async_copy(ex_hbm.at[pl.ds(0, EB)], exb[p],
                                  ge_s[p]).wait()
            pltpu.make_async_copy(xp_hbm.at[sidx[p]], xsb[p],
                                  gx_s[p]).wait()

        def compute(p):
            @plsc.parallel_loop(0, EB, 16, unroll=2)
            def _(e):
                d16 = didx[p][pl.ds(e, 16)]
                iv = plsc.load_gather(tabi, [lax.shift_right_logical(d16, 4),
                                             lax.bitwise_and(d16, 15)])
                a16 = exb[p][pl.ds(e, 16)] * iv
                for j in range(16):
                    av = jnp.full((16,), a16[j], _f32)
                    row = xsb[p].at[e + j]
                    for h in range(D2P // 16):
                        row[pl.ds(h * 16, 16)] = row[pl.ds(h * 16, 16)] * av

        def issue_outs(j, p):
            pltpu.async_copy(xsb[p], out_sh.at[didx[p]], so_s[p], add=True)

        def wait_outs(p):
            pltpu.make_async_copy(xsb[p], out_sh.at[didx[p]], so_s[p]).wait()

        _pipeline(_stage_idx(ibt, sidx, didx), issue_gathers, wait_gathers,
                  compute, issue_outs, wait_outs)

        plsc.subcore_barrier()
        pltpu.sync_copy(out_sh.at[pl.ds(row0, ROWS_PER_TILE)],
                        out_hbm.at[c].at[pl.ds(row0, ROWS_PER_TILE)])

    return k(pk, ex2, den2p, xp2)


def kernel(x, edge_index, W1, att_src1, att_dst1, b1, W2, att_src2, att_dst2,
           b2):
    i32 = jnp.int32
    npad_edges = ET_PAD - E - N
    loop = jnp.arange(N, dtype=i32)
    padv = N + (jnp.arange(npad_edges, dtype=i32) % (N_PAD - N))
    src = jnp.concatenate([edge_index[0], loop, padv])
    dst = jnp.concatenate([edge_index[1], loop, padv])
    pk = jnp.stack([src.reshape(NTILE * NBLK, EB),
                    dst.reshape(NTILE * NBLK, EB)], axis=1)

    x_pad = jnp.pad(x, ((0, N_PAD - N), (0, 0)))
    a1s = jnp.pad((att_src1[:, :, None] * jnp.eye(H1, dtype=_f32)[:, None, :])
                  .reshape(D1, H1), ((0, 0), (0, HP - H1)))
    a1d = jnp.pad((att_dst1[:, :, None] * jnp.eye(H1, dtype=_f32)[:, None, :])
                  .reshape(D1, H1), ((0, 0), (0, HP - H1)))
    w2p = jnp.pad(W2, ((0, 0), (0, D2P - C2)))
    a2 = jnp.zeros((D2P, HP), _f32)
    a2 = a2.at[:C2, 0].set(att_src2[0]).at[:C2, 1].set(att_dst2[0])
    b1r = b1.reshape(1, D1)
    b2p = jnp.pad(b2, (0, D2P - C2)).reshape(1, D2P)

    xp1a, xp1b, as1, ad1 = _tc_proj1(x_pad, W1, a1s, a1d)
    ex1, den1p = _sc_pass1_l1(pk, as1, ad1)
    out1pa, out1pb = _sc_pass2_l1(pk, ex1, den1p, xp1a, xp1b)

    xp2, aall = _tc_proj2(out1pa, out1pb, b1r, w2p, a2)
    as2 = aall[:, 0]
    ad2 = aall[:, 1]
    ex2, den2p = _sc_pass1_l2(pk, as2, ad2)
    out2p = _sc_pass2_l2(pk, ex2, den2p, xp2)

    res = _tc_final(out2p, b2p)
    return res[:N, :C2]
